# baseline (speedup 1.0000x reference)
import functools
import math

import jax
import jax.numpy as jnp
import numpy as np
from jax import lax
from jax.experimental import pallas as pl
from jax.experimental.pallas import tpu as pltpu
from jax.experimental.pallas import tpu_sc as plsc

N_NODE = 50000
C = 128
H = 4
DH = 32
BN = 2000

NC = 2
NS = 16
NW = NC * NS
CH = 128
NCHUNK = 74
EPT = NCHUNK * CH
E_PAD = NW * EPT
E_REAL = 300000
NSP = 50048
RPT = NSP // NS


def _proj_relu_body(x_ref, w_ref, b_ref, o_ref):
    y = jnp.dot(x_ref[...], w_ref[...], preferred_element_type=jnp.float32)
    o_ref[...] = jax.nn.relu(y + b_ref[...])


def _proj_relu(x, w, b):
    n = x.shape[0]
    grid = (n // BN,)
    return pl.pallas_call(
        _proj_relu_body,
        grid=grid,
        in_specs=[
            pl.BlockSpec((BN, x.shape[1]), lambda i: (i, 0)),
            pl.BlockSpec((x.shape[1], w.shape[1]), lambda i: (0, 0)),
            pl.BlockSpec((1, w.shape[1]), lambda i: (0, 0)),
        ],
        out_specs=pl.BlockSpec((BN, w.shape[1]), lambda i: (i, 0)),
        out_shape=jax.ShapeDtypeStruct((n, w.shape[1]), jnp.float32),
    )(x, w, b.reshape(1, -1))


def _proj_body(x_ref, w_ref, o_ref):
    o_ref[...] = jnp.dot(x_ref[...], w_ref[...], preferred_element_type=jnp.float32)


def _proj(x, w):
    n = x.shape[0]
    grid = (n // BN,)
    return pl.pallas_call(
        _proj_body,
        grid=grid,
        in_specs=[
            pl.BlockSpec((BN, x.shape[1]), lambda i: (i, 0)),
            pl.BlockSpec((x.shape[1], w.shape[1]), lambda i: (0, 0)),
        ],
        out_specs=pl.BlockSpec((BN, w.shape[1]), lambda i: (i, 0)),
        out_shape=jax.ShapeDtypeStruct((n, w.shape[1]), jnp.float32),
    )(x, w)


def _out_stage_body(c_ref, m_ref, h_ref, w_ref, b_ref, o_ref):
    o = jax.nn.gelu(m_ref[...])
    o = jnp.dot(o, w_ref[...], preferred_element_type=jnp.float32) + b_ref[...]
    o_ref[...] = c_ref[0] * o + c_ref[1] * h_ref[...]


def _out_stage(m, h, w, b, beta, hcoef):
    n = m.shape[0]
    grid = (n // BN,)
    coef = jnp.stack([beta, hcoef]).astype(jnp.float32)
    return pl.pallas_call(
        _out_stage_body,
        grid=grid,
        in_specs=[
            pl.BlockSpec(memory_space=pltpu.SMEM),
            pl.BlockSpec((BN, C), lambda i: (i, 0)),
            pl.BlockSpec((BN, C), lambda i: (i, 0)),
            pl.BlockSpec((C, C), lambda i: (0, 0)),
            pl.BlockSpec((1, C), lambda i: (0, 0)),
        ],
        out_specs=pl.BlockSpec((BN, C), lambda i: (i, 0)),
        out_shape=jax.ShapeDtypeStruct((n, C), jnp.float32),
    )(coef, m, h, w, b.reshape(1, -1))


def _pass1_body(q_hbm, krel_hbm, sidx_hbm, didx_hbm,
                ex_hbm, den0_hbm, den1_hbm,
                sidx_v, didx_v, ke_v, qe_v, ex_v, den_sp, sem1, sem2):
    cid = lax.axis_index("c")
    sid = lax.axis_index("s")
    wid = cid * NS + sid
    iot = lax.iota(jnp.int32, 16)
    zero16 = jnp.zeros((16,), jnp.float32)

    for r in range(CH):
        ex_v[r] = zero16
    row0 = sid * RPT
    for j in range(RPT // CH):
        pltpu.sync_copy(ex_v, den_sp.at[pl.ds(row0 + j * CH, CH), :])
    rem = RPT - (RPT // CH) * CH
    if rem:
        pltpu.sync_copy(ex_v.at[pl.ds(0, rem), :],
                        den_sp.at[pl.ds(row0 + (RPT // CH) * CH, rem), :])
    plsc.subcore_barrier()

    def chunk_body(c, carry):
        base = (wid * NCHUNK + c) * CH
        pltpu.sync_copy(sidx_hbm.at[pl.ds(base, CH)], sidx_v)
        pltpu.sync_copy(didx_hbm.at[pl.ds(base, CH)], didx_v)
        cp1 = pltpu.async_copy(krel_hbm.at[sidx_v], ke_v, sem1)
        cp2 = pltpu.async_copy(q_hbm.at[didx_v], qe_v, sem2)
        cp1.wait()
        cp2.wait()
        for g in range(CH // 16):
            rows = iot + g * 16
            for h in range(H):
                def dd_body(dd, acc):
                    colv = jnp.full((16,), h * DH + dd, jnp.int32)
                    kv = plsc.load_gather(ke_v, [rows, colv])
                    qv = plsc.load_gather(qe_v, [rows, colv])
                    return acc + kv * qv
                acc_h = lax.fori_loop(0, DH, dd_body, zero16, unroll=8)
                mask = (base + g * 16 + iot) < E_REAL
                exh = jnp.where(mask, jnp.exp(acc_h), 0.0)
                plsc.store_scatter(ex_v, [rows, jnp.full((16,), h, jnp.int32)], exh)
        pltpu.sync_copy(ex_v, ex_hbm.at[pl.ds(base, CH), :])
        pltpu.sync_copy(ex_v, den_sp.at[didx_v], add=True)
        return carry

    lax.fori_loop(0, NCHUNK, chunk_body, 0)
    plsc.subcore_barrier()

    @pl.when(cid == 0)
    def _():
        pltpu.sync_copy(den_sp.at[pl.ds(row0, RPT), :],
                        den0_hbm.at[pl.ds(row0, RPT), :])

    @pl.when(cid == 1)
    def _():
        pltpu.sync_copy(den_sp.at[pl.ds(row0, RPT), :],
                        den1_hbm.at[pl.ds(row0, RPT), :])


def _sc_pass1(q_dst, krel_src, s_pad, d_pad):
    mesh = plsc.VectorSubcoreMesh(core_axis_name="c", subcore_axis_name="s",
                                  num_cores=NC, num_subcores=NS)
    fn = pl.kernel(
        _pass1_body,
        out_type=[
            jax.ShapeDtypeStruct((E_PAD, 16), jnp.float32),
            jax.ShapeDtypeStruct((NSP, 16), jnp.float32),
            jax.ShapeDtypeStruct((NSP, 16), jnp.float32),
        ],
        mesh=mesh,
        compiler_params=pltpu.CompilerParams(
            use_tc_tiling_on_sc=False, needs_layout_passes=False),
        scratch_types=[
            pltpu.VMEM((CH,), jnp.int32),
            pltpu.VMEM((CH,), jnp.int32),
            pltpu.VMEM((CH, C), jnp.float32),
            pltpu.VMEM((CH, C), jnp.float32),
            pltpu.VMEM((CH, 16), jnp.float32),
            pltpu.VMEM_SHARED((NSP, 16), jnp.float32),
            pltpu.SemaphoreType.DMA,
            pltpu.SemaphoreType.DMA,
        ],
    )
    return fn(q_dst, krel_src, s_pad, d_pad)


def _pass2_body(vrelp_hbm, sidx_hbm, didx_hbm, ex_hbm, den0_hbm, den1_hbm,
                out0_hbm, out1_hbm,
                sidx_v, didx_v, pidx_v, moff_v, d0_v, d1_v, exb_v,
                vb_v, msg_v, a_sp, out_sp, sem1, sem2):
    cid = lax.axis_index("c")
    sid = lax.axis_index("s")
    wid = cid * NS + sid
    iot = lax.iota(jnp.int32, 16)
    zero16 = jnp.zeros((16,), jnp.float32)
    row0 = sid * RPT

    def chunkA(c, carry):
        base = (wid * NCHUNK + c) * CH
        pltpu.sync_copy(didx_hbm.at[pl.ds(base, CH)], didx_v)
        cp1 = pltpu.async_copy(den0_hbm.at[didx_v], d0_v, sem1)
        cp2 = pltpu.async_copy(den1_hbm.at[didx_v], d1_v, sem2)
        pltpu.sync_copy(ex_hbm.at[pl.ds(base, CH), :], exb_v)
        cp1.wait()
        cp2.wait()
        for g in range(CH // 16):
            rows16 = iot + g * 16
            eidx = (c * CH + g * 16 + iot) * 4
            for h in range(H):
                hv = jnp.full((16,), h, jnp.int32)
                exh = plsc.load_gather(exb_v, [rows16, hv])
                den = (plsc.load_gather(d0_v, [rows16, hv])
                       + plsc.load_gather(d1_v, [rows16, hv]))
                plsc.store_scatter(a_sp, [eidx + h], exh / (den + 1e-16))
        return carry

    lax.fori_loop(0, NCHUNK, chunkA, 0)

    def head_block(bb, carry):
        for r in range(CH):
            msg_v[r] = zero16
        for j in range(RPT // CH):
            pltpu.sync_copy(msg_v, out_sp.at[pl.ds(row0 + j * CH, CH), :])
        rem = RPT - (RPT // CH) * CH
        if rem:
            pltpu.sync_copy(msg_v.at[pl.ds(0, rem), :],
                            out_sp.at[pl.ds(row0 + (RPT // CH) * CH, rem), :])
        plsc.subcore_barrier()

        def chunkB(c, carry2):
            base = (wid * NCHUNK + c) * CH
            pltpu.sync_copy(sidx_hbm.at[pl.ds(base, CH)], sidx_v)
            pltpu.sync_copy(didx_hbm.at[pl.ds(base, CH)], didx_v)
            for g in range(CH // 16):
                sv = sidx_v[pl.ds(g * 16, 16)]
                pidx_v[pl.ds(g * 16, 16)] = (sv >> 3) + bb * (N_NODE // 8)
                moff_v[pl.ds(g * 16, 16)] = (sv & 7) << 4
            pltpu.async_copy(vrelp_hbm.at[pidx_v], vb_v, sem1).wait()
            for g in range(CH // 16):
                rows16 = iot + g * 16
                mo16 = moff_v[pl.ds(g * 16, 16)]
                aidx = (c * CH + g * 16 + iot) * 4 + (bb >> 1)
                av16 = plsc.load_gather(a_sp, [aidx])
                for j in range(16):
                    val = plsc.load_gather(vb_v, [rows16, mo16 + j]) * av16
                    plsc.store_scatter(msg_v, [rows16, jnp.full((16,), j, jnp.int32)], val)
            pltpu.sync_copy(msg_v, out_sp.at[didx_v], add=True)
            return carry2

        lax.fori_loop(0, NCHUNK, chunkB, 0)
        plsc.subcore_barrier()

        @pl.when(cid == 0)
        def _():
            pltpu.sync_copy(out_sp.at[pl.ds(row0, RPT), :],
                            out0_hbm.at[pl.ds(bb * NSP + row0, RPT), :])

        @pl.when(cid == 1)
        def _():
            pltpu.sync_copy(out_sp.at[pl.ds(row0, RPT), :],
                            out1_hbm.at[pl.ds(bb * NSP + row0, RPT), :])
        plsc.subcore_barrier()
        return carry

    lax.fori_loop(0, 2 * H, head_block, 0)


def _sc_pass2(vrel_pack, s_pad, d_pad, ex, den0, den1):
    mesh = plsc.VectorSubcoreMesh(core_axis_name="c", subcore_axis_name="s",
                                  num_cores=NC, num_subcores=NS)
    fn = pl.kernel(
        _pass2_body,
        out_type=[
            jax.ShapeDtypeStruct((2 * H * NSP, 16), jnp.float32),
            jax.ShapeDtypeStruct((2 * H * NSP, 16), jnp.float32),
        ],
        mesh=mesh,
        compiler_params=pltpu.CompilerParams(
            use_tc_tiling_on_sc=False, needs_layout_passes=False),
        scratch_types=[
            pltpu.VMEM((CH,), jnp.int32),
            pltpu.VMEM((CH,), jnp.int32),
            pltpu.VMEM((CH,), jnp.int32),
            pltpu.VMEM((CH,), jnp.int32),
            pltpu.VMEM((CH, 16), jnp.float32),
            pltpu.VMEM((CH, 16), jnp.float32),
            pltpu.VMEM((CH, 16), jnp.float32),
            pltpu.VMEM((CH, C), jnp.float32),
            pltpu.VMEM((CH, 16), jnp.float32),
            pltpu.VMEM((EPT * 4,), jnp.float32),
            pltpu.VMEM_SHARED((NSP, 16), jnp.float32),
            pltpu.SemaphoreType.DMA,
            pltpu.SemaphoreType.DMA,
        ],
    )
    return fn(vrel_pack, s_pad, d_pad, ex, den0, den1)


def _edge_phase(q_dst, krel_src, vrel_src, s, d, n_dst):
    pad = jnp.arange(E_PAD - E_REAL, dtype=jnp.int32) % N_NODE
    s_pad = jnp.concatenate([s.astype(jnp.int32), pad])
    d_pad = jnp.concatenate([d.astype(jnp.int32), pad])
    ex, den0, den1 = _sc_pass1(q_dst, krel_src, s_pad, d_pad)
    vrel_pack = jnp.concatenate(
        [vrel_src[:, bb * 16:(bb + 1) * 16].reshape(N_NODE // 8, C)
         for bb in range(2 * H)], axis=0)
    o0, o1 = _sc_pass2(vrel_pack, s_pad, d_pad, ex, den0, den1)
    osum = (o0 + o1).reshape(2 * H, NSP, 16)[:, :n_dst, :]
    return osum.transpose(1, 0, 2).reshape(n_dst, C)


_SRC_EDGE = {'user': 'ui', 'item': 'iu'}
_EDGE_DEFS = (('ui', 'user', 'item'), ('iu', 'item', 'user'))


def _fold_params(params):
    folded = {}
    inv_sqrt = 1.0 / math.sqrt(float(DH))
    for l in range(2):
        for t in ('user', 'item'):
            e = _SRC_EDGE[t]
            arel = params['l%d_arel_%s' % (l, e)]
            mrel = params['l%d_mrel_%s' % (l, e)]
            prel = params['l%d_prel_%s' % (l, e)] * inv_sqrt
            Wk = params['l%d_Wk_%s' % (l, t)].reshape(C, H, DH)
            Wv = params['l%d_Wv_%s' % (l, t)].reshape(C, H, DH)
            Wk_f = jnp.einsum('chd,hde,h->che', Wk, arel, prel).reshape(C, C)
            Wv_f = jnp.einsum('chd,hde->che', Wv, mrel).reshape(C, C)
            Wq = params['l%d_Wq_%s' % (l, t)]
            folded['Wqkv_%d_%s' % (l, t)] = jnp.concatenate([Wq, Wk_f, Wv_f], axis=1)
    return folded


def kernel(x_user, x_item, edge_index_user_item, edge_index_item_user, params):
    folded = _fold_params(params)
    h = {'user': _proj_relu(x_user, params['in_W_user'], params['in_b_user']),
         'item': _proj_relu(x_item, params['in_W_item'], params['in_b_item'])}
    ei = {'ui': (edge_index_user_item[0], edge_index_user_item[1]),
          'iu': (edge_index_item_user[0], edge_index_item_user[1])}
    for l in range(2):
        q, krel, vrel = {}, {}, {}
        for t in h:
            y = _proj(h[t], folded['Wqkv_%d_%s' % (l, t)])
            q[t] = y[:, :C]
            krel[t] = y[:, C:2 * C]
            vrel[t] = y[:, 2 * C:]
        out = {}
        for e, src, dst in _EDGE_DEFS:
            s, d = ei[e]
            out[dst] = _edge_phase(q[dst], krel[src], vrel[src], s, d, N_NODE)
        h_new = {}
        for t in h:
            beta = jax.nn.sigmoid(params['l%d_skip_%s' % (l, t)])
            hcoef = (1.0 - beta) + (1.0 if l > 0 else 0.0)
            h_new[t] = _out_stage(out[t], h[t],
                                  params['l%d_Wa_%s' % (l, t)],
                                  params['l%d_ba_%s' % (l, t)], beta, hcoef)
        h = h_new
    return (h['user'], h['item'])


# speedup vs baseline: 1.1007x; 1.1007x over previous
import functools
import math

import jax
import jax.numpy as jnp
import numpy as np
from jax import lax
from jax.experimental import pallas as pl
from jax.experimental.pallas import tpu as pltpu
from jax.experimental.pallas import tpu_sc as plsc

N_NODE = 50000
C = 128
H = 4
DH = 32
BN = 2000

NC = 2
NS = 16
NW = NC * NS
CH = 128
NCHUNK = 74
EPT = NCHUNK * CH
E_PAD = NW * EPT
E_REAL = 300000
NSP = 50048
RPT = NSP // NS


def _proj_relu_body(x_ref, w_ref, b_ref, o_ref):
    y = jnp.dot(x_ref[...], w_ref[...], preferred_element_type=jnp.float32)
    o_ref[...] = jax.nn.relu(y + b_ref[...])


def _proj_relu(x, w, b):
    n = x.shape[0]
    grid = (n // BN,)
    return pl.pallas_call(
        _proj_relu_body,
        grid=grid,
        in_specs=[
            pl.BlockSpec((BN, x.shape[1]), lambda i: (i, 0)),
            pl.BlockSpec((x.shape[1], w.shape[1]), lambda i: (0, 0)),
            pl.BlockSpec((1, w.shape[1]), lambda i: (0, 0)),
        ],
        out_specs=pl.BlockSpec((BN, w.shape[1]), lambda i: (i, 0)),
        out_shape=jax.ShapeDtypeStruct((n, w.shape[1]), jnp.float32),
    )(x, w, b.reshape(1, -1))


def _proj_body(x_ref, w_ref, o_ref):
    o_ref[...] = jnp.dot(x_ref[...], w_ref[...], preferred_element_type=jnp.float32)


def _proj(x, w):
    n = x.shape[0]
    grid = (n // BN,)
    return pl.pallas_call(
        _proj_body,
        grid=grid,
        in_specs=[
            pl.BlockSpec((BN, x.shape[1]), lambda i: (i, 0)),
            pl.BlockSpec((x.shape[1], w.shape[1]), lambda i: (0, 0)),
        ],
        out_specs=pl.BlockSpec((BN, w.shape[1]), lambda i: (i, 0)),
        out_shape=jax.ShapeDtypeStruct((n, w.shape[1]), jnp.float32),
    )(x, w)


def _out_stage_body(c_ref, m_ref, h_ref, w_ref, b_ref, o_ref):
    o = jax.nn.gelu(m_ref[...])
    o = jnp.dot(o, w_ref[...], preferred_element_type=jnp.float32) + b_ref[...]
    o_ref[...] = c_ref[0] * o + c_ref[1] * h_ref[...]


def _out_stage(m, h, w, b, beta, hcoef):
    n = m.shape[0]
    grid = (n // BN,)
    coef = jnp.stack([beta, hcoef]).astype(jnp.float32)
    return pl.pallas_call(
        _out_stage_body,
        grid=grid,
        in_specs=[
            pl.BlockSpec(memory_space=pltpu.SMEM),
            pl.BlockSpec((BN, C), lambda i: (i, 0)),
            pl.BlockSpec((BN, C), lambda i: (i, 0)),
            pl.BlockSpec((C, C), lambda i: (0, 0)),
            pl.BlockSpec((1, C), lambda i: (0, 0)),
        ],
        out_specs=pl.BlockSpec((BN, C), lambda i: (i, 0)),
        out_shape=jax.ShapeDtypeStruct((n, C), jnp.float32),
    )(coef, m, h, w, b.reshape(1, -1))


def _pass1_body(q_hbm, krel_hbm, sidx_hbm, didx_hbm,
                ex_hbm, den0_hbm, den1_hbm,
                sidx_v, didx_v, ke_v, qe_v, ex_v, den_sp, sem1, sem2):
    cid = lax.axis_index("c")
    sid = lax.axis_index("s")
    wid = cid * NS + sid
    iot = lax.iota(jnp.int32, 16)
    zero16 = jnp.zeros((16,), jnp.float32)

    for r in range(CH):
        ex_v[r] = zero16
    row0 = sid * RPT
    for j in range(RPT // CH):
        pltpu.sync_copy(ex_v, den_sp.at[pl.ds(row0 + j * CH, CH), :])
    rem = RPT - (RPT // CH) * CH
    if rem:
        pltpu.sync_copy(ex_v.at[pl.ds(0, rem), :],
                        den_sp.at[pl.ds(row0 + (RPT // CH) * CH, rem), :])
    plsc.subcore_barrier()

    def chunk_body(c, carry):
        base = (wid * NCHUNK + c) * CH
        pltpu.sync_copy(sidx_hbm.at[pl.ds(base, CH)], sidx_v)
        pltpu.sync_copy(didx_hbm.at[pl.ds(base, CH)], didx_v)
        cp1 = pltpu.async_copy(krel_hbm.at[sidx_v], ke_v, sem1)
        cp2 = pltpu.async_copy(q_hbm.at[didx_v], qe_v, sem2)
        cp1.wait()
        cp2.wait()
        for g in range(CH // 16):
            rows = iot + g * 16
            for h in range(H):
                def dd_body(dd, acc):
                    colv = jnp.full((16,), h * DH + dd, jnp.int32)
                    kv = plsc.load_gather(ke_v, [rows, colv])
                    qv = plsc.load_gather(qe_v, [rows, colv])
                    return acc + kv * qv
                acc_h = lax.fori_loop(0, DH, dd_body, zero16, unroll=8)
                mask = (base + g * 16 + iot) < E_REAL
                exh = jnp.where(mask, jnp.exp(acc_h), 0.0)
                plsc.store_scatter(ex_v, [rows, jnp.full((16,), h, jnp.int32)], exh)
        pltpu.sync_copy(ex_v, ex_hbm.at[pl.ds(base, CH), :])
        pltpu.sync_copy(ex_v, den_sp.at[didx_v], add=True)
        return carry

    lax.fori_loop(0, NCHUNK, chunk_body, 0)
    plsc.subcore_barrier()

    @pl.when(cid == 0)
    def _():
        pltpu.sync_copy(den_sp.at[pl.ds(row0, RPT), :],
                        den0_hbm.at[pl.ds(row0, RPT), :])

    @pl.when(cid == 1)
    def _():
        pltpu.sync_copy(den_sp.at[pl.ds(row0, RPT), :],
                        den1_hbm.at[pl.ds(row0, RPT), :])


def _sc_pass1(q_dst, krel_src, s_pad, d_pad):
    mesh = plsc.VectorSubcoreMesh(core_axis_name="c", subcore_axis_name="s",
                                  num_cores=NC, num_subcores=NS)
    fn = pl.kernel(
        _pass1_body,
        out_type=[
            jax.ShapeDtypeStruct((E_PAD, 16), jnp.float32),
            jax.ShapeDtypeStruct((NSP, 16), jnp.float32),
            jax.ShapeDtypeStruct((NSP, 16), jnp.float32),
        ],
        mesh=mesh,
        compiler_params=pltpu.CompilerParams(
            use_tc_tiling_on_sc=False, needs_layout_passes=False),
        scratch_types=[
            pltpu.VMEM((CH,), jnp.int32),
            pltpu.VMEM((CH,), jnp.int32),
            pltpu.VMEM((CH, C), jnp.float32),
            pltpu.VMEM((CH, C), jnp.float32),
            pltpu.VMEM((CH, 16), jnp.float32),
            pltpu.VMEM_SHARED((NSP, 16), jnp.float32),
            pltpu.SemaphoreType.DMA,
            pltpu.SemaphoreType.DMA,
        ],
    )
    return fn(q_dst, krel_src, s_pad, d_pad)


def _pass2_body(vrelp_hbm, sidx_hbm, didx_hbm, ex_hbm, den0_hbm, den1_hbm,
                out0_hbm, out1_hbm,
                sidx_v, didx_v, pidx_v, d0_v, d1_v, exb_v,
                vb_v, msg_v, a_sp, out_sp, sem1, sem2):
    cid = lax.axis_index("c")
    sid = lax.axis_index("s")
    wid = cid * NS + sid
    iot = lax.iota(jnp.int32, 16)
    zero16 = jnp.zeros((16,), jnp.float32)
    row0 = sid * RPT

    def chunkA(c, carry):
        base = (wid * NCHUNK + c) * CH
        pltpu.sync_copy(didx_hbm.at[pl.ds(base, CH)], didx_v)
        cp1 = pltpu.async_copy(den0_hbm.at[didx_v], d0_v, sem1)
        cp2 = pltpu.async_copy(den1_hbm.at[didx_v], d1_v, sem2)
        pltpu.sync_copy(ex_hbm.at[pl.ds(base, CH), :], exb_v)
        cp1.wait()
        cp2.wait()
        for g in range(CH // 16):
            rows16 = iot + g * 16
            eidx = (c * CH + g * 16 + iot) * 4
            for h in range(H):
                hv = jnp.full((16,), h, jnp.int32)
                exh = plsc.load_gather(exb_v, [rows16, hv])
                den = (plsc.load_gather(d0_v, [rows16, hv])
                       + plsc.load_gather(d1_v, [rows16, hv]))
                plsc.store_scatter(a_sp, [eidx + h], exh / (den + 1e-16))
        return carry

    lax.fori_loop(0, NCHUNK, chunkA, 0)

    def head_block(bb, carry):
        for r in range(CH):
            msg_v[r] = zero16
        for j in range(RPT // CH):
            pltpu.sync_copy(msg_v, out_sp.at[pl.ds(row0 + j * CH, CH), :])
        rem = RPT - (RPT // CH) * CH
        if rem:
            pltpu.sync_copy(msg_v.at[pl.ds(0, rem), :],
                            out_sp.at[pl.ds(row0 + (RPT // CH) * CH, rem), :])
        plsc.subcore_barrier()

        def chunkB(c, carry2):
            base = (wid * NCHUNK + c) * CH
            pltpu.sync_copy(sidx_hbm.at[pl.ds(base, CH)], sidx_v)
            pltpu.sync_copy(didx_hbm.at[pl.ds(base, CH)], didx_v)
            for g in range(CH // 16):
                sv = sidx_v[pl.ds(g * 16, 16)]
                pidx_v[pl.ds(g * 16, 16)] = sv + bb * N_NODE
            pltpu.async_copy(vrelp_hbm.at[pidx_v], vb_v, sem1).wait()
            for g in range(CH // 16):
                rows16 = iot + g * 16
                aidx = (c * CH + g * 16 + iot) * 4 + (bb >> 1)
                av16 = plsc.load_gather(a_sp, [aidx])
                for j in range(16):
                    jv = jnp.full((16,), j, jnp.int32)
                    val = plsc.load_gather(vb_v, [rows16, jv]) * av16
                    plsc.store_scatter(msg_v, [rows16, jv], val)
            pltpu.sync_copy(msg_v, out_sp.at[didx_v], add=True)
            return carry2

        lax.fori_loop(0, NCHUNK, chunkB, 0)
        plsc.subcore_barrier()

        @pl.when(cid == 0)
        def _():
            pltpu.sync_copy(out_sp.at[pl.ds(row0, RPT), :],
                            out0_hbm.at[pl.ds(bb * NSP + row0, RPT), :])

        @pl.when(cid == 1)
        def _():
            pltpu.sync_copy(out_sp.at[pl.ds(row0, RPT), :],
                            out1_hbm.at[pl.ds(bb * NSP + row0, RPT), :])
        plsc.subcore_barrier()
        return carry

    lax.fori_loop(0, 2 * H, head_block, 0)


def _sc_pass2(vrel_pack, s_pad, d_pad, ex, den0, den1):
    mesh = plsc.VectorSubcoreMesh(core_axis_name="c", subcore_axis_name="s",
                                  num_cores=NC, num_subcores=NS)
    fn = pl.kernel(
        _pass2_body,
        out_type=[
            jax.ShapeDtypeStruct((2 * H * NSP, 16), jnp.float32),
            jax.ShapeDtypeStruct((2 * H * NSP, 16), jnp.float32),
        ],
        mesh=mesh,
        compiler_params=pltpu.CompilerParams(
            use_tc_tiling_on_sc=False, needs_layout_passes=False),
        scratch_types=[
            pltpu.VMEM((CH,), jnp.int32),
            pltpu.VMEM((CH,), jnp.int32),
            pltpu.VMEM((CH,), jnp.int32),
            pltpu.VMEM((CH, 16), jnp.float32),
            pltpu.VMEM((CH, 16), jnp.float32),
            pltpu.VMEM((CH, 16), jnp.float32),
            pltpu.VMEM((CH, 16), jnp.float32),
            pltpu.VMEM((CH, 16), jnp.float32),
            pltpu.VMEM((EPT * 4,), jnp.float32),
            pltpu.VMEM_SHARED((NSP, 16), jnp.float32),
            pltpu.SemaphoreType.DMA,
            pltpu.SemaphoreType.DMA,
        ],
    )
    return fn(vrel_pack, s_pad, d_pad, ex, den0, den1)


def _edge_phase(q_dst, krel_src, vrel_src, s, d, n_dst):
    pad = jnp.arange(E_PAD - E_REAL, dtype=jnp.int32) % N_NODE
    s_pad = jnp.concatenate([s.astype(jnp.int32), pad])
    d_pad = jnp.concatenate([d.astype(jnp.int32), pad])
    ex, den0, den1 = _sc_pass1(q_dst, krel_src, s_pad, d_pad)
    vrel_pack = vrel_src.reshape(N_NODE, 2 * H, 16).transpose(1, 0, 2).reshape(
        2 * H * N_NODE, 16)
    o0, o1 = _sc_pass2(vrel_pack, s_pad, d_pad, ex, den0, den1)
    osum = (o0 + o1).reshape(2 * H, NSP, 16)[:, :n_dst, :]
    return osum.transpose(1, 0, 2).reshape(n_dst, C)


_SRC_EDGE = {'user': 'ui', 'item': 'iu'}
_EDGE_DEFS = (('ui', 'user', 'item'), ('iu', 'item', 'user'))


def _fold_params(params):
    folded = {}
    inv_sqrt = 1.0 / math.sqrt(float(DH))
    for l in range(2):
        for t in ('user', 'item'):
            e = _SRC_EDGE[t]
            arel = params['l%d_arel_%s' % (l, e)]
            mrel = params['l%d_mrel_%s' % (l, e)]
            prel = params['l%d_prel_%s' % (l, e)] * inv_sqrt
            Wk = params['l%d_Wk_%s' % (l, t)].reshape(C, H, DH)
            Wv = params['l%d_Wv_%s' % (l, t)].reshape(C, H, DH)
            Wk_f = jnp.einsum('chd,hde,h->che', Wk, arel, prel).reshape(C, C)
            Wv_f = jnp.einsum('chd,hde->che', Wv, mrel).reshape(C, C)
            Wq = params['l%d_Wq_%s' % (l, t)]
            folded['Wqkv_%d_%s' % (l, t)] = jnp.concatenate([Wq, Wk_f, Wv_f], axis=1)
    return folded


def kernel(x_user, x_item, edge_index_user_item, edge_index_item_user, params):
    folded = _fold_params(params)
    h = {'user': _proj_relu(x_user, params['in_W_user'], params['in_b_user']),
         'item': _proj_relu(x_item, params['in_W_item'], params['in_b_item'])}
    ei = {'ui': (edge_index_user_item[0], edge_index_user_item[1]),
          'iu': (edge_index_item_user[0], edge_index_item_user[1])}
    for l in range(2):
        q, krel, vrel = {}, {}, {}
        for t in h:
            y = _proj(h[t], folded['Wqkv_%d_%s' % (l, t)])
            q[t] = y[:, :C]
            krel[t] = y[:, C:2 * C]
            vrel[t] = y[:, 2 * C:]
        out = {}
        for e, src, dst in _EDGE_DEFS:
            s, d = ei[e]
            out[dst] = _edge_phase(q[dst], krel[src], vrel[src], s, d, N_NODE)
        h_new = {}
        for t in h:
            beta = jax.nn.sigmoid(params['l%d_skip_%s' % (l, t)])
            hcoef = (1.0 - beta) + (1.0 if l > 0 else 0.0)
            h_new[t] = _out_stage(out[t], h[t],
                                  params['l%d_Wa_%s' % (l, t)],
                                  params['l%d_ba_%s' % (l, t)], beta, hcoef)
        h = h_new
    return (h['user'], h['item'])


# speedup vs baseline: 1.2460x; 1.1320x over previous
import functools
import math

import jax
import jax.numpy as jnp
import numpy as np
from jax import lax
from jax.experimental import pallas as pl
from jax.experimental.pallas import tpu as pltpu
from jax.experimental.pallas import tpu_sc as plsc

N_NODE = 50000
C = 128
H = 4
DH = 32
BN = 2000

NC = 2
NS = 16
NW = NC * NS
CH = 128
NCHUNK = 74
EPT = NCHUNK * CH
E_PAD = NW * EPT
E_REAL = 300000
NSP = 50048
RPT = NSP // NS


def _proj_relu_body(x_ref, w_ref, b_ref, o_ref):
    y = jnp.dot(x_ref[...], w_ref[...], preferred_element_type=jnp.float32)
    o_ref[...] = jax.nn.relu(y + b_ref[...])


def _proj_relu(x, w, b):
    n = x.shape[0]
    grid = (n // BN,)
    return pl.pallas_call(
        _proj_relu_body,
        grid=grid,
        in_specs=[
            pl.BlockSpec((BN, x.shape[1]), lambda i: (i, 0)),
            pl.BlockSpec((x.shape[1], w.shape[1]), lambda i: (0, 0)),
            pl.BlockSpec((1, w.shape[1]), lambda i: (0, 0)),
        ],
        out_specs=pl.BlockSpec((BN, w.shape[1]), lambda i: (i, 0)),
        out_shape=jax.ShapeDtypeStruct((n, w.shape[1]), jnp.float32),
    )(x, w, b.reshape(1, -1))


def _proj_body(x_ref, w_ref, o_ref):
    o_ref[...] = jnp.dot(x_ref[...], w_ref[...], preferred_element_type=jnp.float32)


def _proj(x, w):
    n = x.shape[0]
    grid = (n // BN,)
    return pl.pallas_call(
        _proj_body,
        grid=grid,
        in_specs=[
            pl.BlockSpec((BN, x.shape[1]), lambda i: (i, 0)),
            pl.BlockSpec((x.shape[1], w.shape[1]), lambda i: (0, 0)),
        ],
        out_specs=pl.BlockSpec((BN, w.shape[1]), lambda i: (i, 0)),
        out_shape=jax.ShapeDtypeStruct((n, w.shape[1]), jnp.float32),
    )(x, w)


def _out_stage_body(c_ref, m_ref, h_ref, w_ref, b_ref, o_ref):
    o = jax.nn.gelu(m_ref[...])
    o = jnp.dot(o, w_ref[...], preferred_element_type=jnp.float32) + b_ref[...]
    o_ref[...] = c_ref[0] * o + c_ref[1] * h_ref[...]


def _out_stage(m, h, w, b, beta, hcoef):
    n = m.shape[0]
    grid = (n // BN,)
    coef = jnp.stack([beta, hcoef]).astype(jnp.float32)
    return pl.pallas_call(
        _out_stage_body,
        grid=grid,
        in_specs=[
            pl.BlockSpec(memory_space=pltpu.SMEM),
            pl.BlockSpec((BN, C), lambda i: (i, 0)),
            pl.BlockSpec((BN, C), lambda i: (i, 0)),
            pl.BlockSpec((C, C), lambda i: (0, 0)),
            pl.BlockSpec((1, C), lambda i: (0, 0)),
        ],
        out_specs=pl.BlockSpec((BN, C), lambda i: (i, 0)),
        out_shape=jax.ShapeDtypeStruct((n, C), jnp.float32),
    )(coef, m, h, w, b.reshape(1, -1))


def _pass1_body(q_hbm, krel_hbm, sidx_hbm, didx_hbm,
                ex_hbm, den0_hbm, den1_hbm,
                s0_v, s1_v, d0_v, d1_v, ke0_v, ke1_v, qe0_v, qe1_v,
                ex_v, den_sp, semk0, semq0, semk1, semq1):
    cid = lax.axis_index("c")
    sid = lax.axis_index("s")
    wid = cid * NS + sid
    iot = lax.iota(jnp.int32, 16)
    zero16 = jnp.zeros((16,), jnp.float32)
    sbuf = [s0_v, s1_v]
    dbuf = [d0_v, d1_v]
    kbuf = [ke0_v, ke1_v]
    qbuf = [qe0_v, qe1_v]
    ksem = [semk0, semk1]
    qsem = [semq0, semq1]

    for r in range(CH):
        ex_v[r] = zero16
    row0 = sid * RPT
    for j in range(RPT // CH):
        pltpu.sync_copy(ex_v, den_sp.at[pl.ds(row0 + j * CH, CH), :])
    rem = RPT - (RPT // CH) * CH
    if rem:
        pltpu.sync_copy(ex_v.at[pl.ds(0, rem), :],
                        den_sp.at[pl.ds(row0 + (RPT // CH) * CH, rem), :])
    plsc.subcore_barrier()

    def issue(c, b):
        base = (wid * NCHUNK + c) * CH
        pltpu.sync_copy(sidx_hbm.at[pl.ds(base, CH)], sbuf[b])
        pltpu.sync_copy(didx_hbm.at[pl.ds(base, CH)], dbuf[b])
        pltpu.async_copy(krel_hbm.at[sbuf[b]], kbuf[b], ksem[b])
        pltpu.async_copy(q_hbm.at[dbuf[b]], qbuf[b], qsem[b])

    def compute(c, b):
        pltpu.make_async_copy(krel_hbm.at[sbuf[b]], kbuf[b], ksem[b]).wait()
        pltpu.make_async_copy(q_hbm.at[dbuf[b]], qbuf[b], qsem[b]).wait()
        base = (wid * NCHUNK + c) * CH
        ke, qe = kbuf[b], qbuf[b]
        for g in range(CH // 16):
            rows = iot + g * 16
            for h in range(H):
                def dd_body(dd, acc):
                    colv = jnp.full((16,), h * DH + dd, jnp.int32)
                    kv = plsc.load_gather(ke, [rows, colv])
                    qv = plsc.load_gather(qe, [rows, colv])
                    return acc + kv * qv
                acc_h = lax.fori_loop(0, DH, dd_body, zero16, unroll=16)
                mask = (base + g * 16 + iot) < E_REAL
                exh = jnp.where(mask, jnp.exp(acc_h), 0.0)
                plsc.store_scatter(ex_v, [rows, jnp.full((16,), h, jnp.int32)], exh)
        pltpu.sync_copy(ex_v, ex_hbm.at[pl.ds(base, CH), :])
        pltpu.sync_copy(ex_v, den_sp.at[dbuf[b]], add=True)

    issue(0, 0)

    def pair_body(p, carry):
        c0 = p * 2
        issue(c0 + 1, 1)
        compute(c0, 0)

        @pl.when(c0 + 2 < NCHUNK)
        def _():
            issue(c0 + 2, 0)
        compute(c0 + 1, 1)
        return carry

    lax.fori_loop(0, NCHUNK // 2, pair_body, 0)
    plsc.subcore_barrier()

    @pl.when(cid == 0)
    def _():
        pltpu.sync_copy(den_sp.at[pl.ds(row0, RPT), :],
                        den0_hbm.at[pl.ds(row0, RPT), :])

    @pl.when(cid == 1)
    def _():
        pltpu.sync_copy(den_sp.at[pl.ds(row0, RPT), :],
                        den1_hbm.at[pl.ds(row0, RPT), :])


def _sc_pass1(q_dst, krel_src, s_pad, d_pad):
    mesh = plsc.VectorSubcoreMesh(core_axis_name="c", subcore_axis_name="s",
                                  num_cores=NC, num_subcores=NS)
    fn = pl.kernel(
        _pass1_body,
        out_type=[
            jax.ShapeDtypeStruct((E_PAD, 16), jnp.float32),
            jax.ShapeDtypeStruct((NSP, 16), jnp.float32),
            jax.ShapeDtypeStruct((NSP, 16), jnp.float32),
        ],
        mesh=mesh,
        compiler_params=pltpu.CompilerParams(
            use_tc_tiling_on_sc=False, needs_layout_passes=False),
        scratch_types=[
            pltpu.VMEM((CH,), jnp.int32),
            pltpu.VMEM((CH,), jnp.int32),
            pltpu.VMEM((CH,), jnp.int32),
            pltpu.VMEM((CH,), jnp.int32),
            pltpu.VMEM((CH, C), jnp.float32),
            pltpu.VMEM((CH, C), jnp.float32),
            pltpu.VMEM((CH, C), jnp.float32),
            pltpu.VMEM((CH, C), jnp.float32),
            pltpu.VMEM((CH, 16), jnp.float32),
            pltpu.VMEM_SHARED((NSP, 16), jnp.float32),
            pltpu.SemaphoreType.DMA,
            pltpu.SemaphoreType.DMA,
            pltpu.SemaphoreType.DMA,
            pltpu.SemaphoreType.DMA,
        ],
    )
    return fn(q_dst, krel_src, s_pad, d_pad)


def _pass2_body(vrelp_hbm, sidx_hbm, didx_hbm, ex_hbm, den0_hbm, den1_hbm,
                out0_hbm, out1_hbm,
                ia0_v, ia1_v, ib0_v, ib1_v, ic0_v, ic1_v,
                f0_v, f1_v, f2_v, f3_v, f4_v, f5_v,
                a_sp, out_sp, sm0, sm1, sm2, sm3):
    cid = lax.axis_index("c")
    sid = lax.axis_index("s")
    wid = cid * NS + sid
    iot = lax.iota(jnp.int32, 16)
    zero16 = jnp.zeros((16,), jnp.float32)
    row0 = sid * RPT

    dA = [ia0_v, ia1_v]
    g0 = [f0_v, f1_v]
    g1 = [f2_v, f3_v]
    exb = [f4_v, f5_v]
    s0 = [sm0, sm1]
    s1 = [sm2, sm3]

    def issueA(c, b):
        base = (wid * NCHUNK + c) * CH
        pltpu.sync_copy(didx_hbm.at[pl.ds(base, CH)], dA[b])
        pltpu.async_copy(den0_hbm.at[dA[b]], g0[b], s0[b])
        pltpu.async_copy(den1_hbm.at[dA[b]], g1[b], s1[b])
        pltpu.sync_copy(ex_hbm.at[pl.ds(base, CH), :], exb[b])

    def computeA(c, b):
        pltpu.make_async_copy(den0_hbm.at[dA[b]], g0[b], s0[b]).wait()
        pltpu.make_async_copy(den1_hbm.at[dA[b]], g1[b], s1[b]).wait()
        for g in range(CH // 16):
            rows16 = iot + g * 16
            eidx = (c * CH + g * 16 + iot) * 4
            for h in range(H):
                hv = jnp.full((16,), h, jnp.int32)
                exh = plsc.load_gather(exb[b], [rows16, hv])
                den = (plsc.load_gather(g0[b], [rows16, hv])
                       + plsc.load_gather(g1[b], [rows16, hv]))
                plsc.store_scatter(a_sp, [eidx + h], exh / (den + 1e-16))

    issueA(0, 0)

    def pairA(p, carry):
        c0 = p * 2
        issueA(c0 + 1, 1)
        computeA(c0, 0)

        @pl.when(c0 + 2 < NCHUNK)
        def _():
            issueA(c0 + 2, 0)
        computeA(c0 + 1, 1)
        return carry

    lax.fori_loop(0, NCHUNK // 2, pairA, 0)

    sB = [ia0_v, ia1_v]
    dB = [ib0_v, ib1_v]
    pB = [ic0_v, ic1_v]
    vB = [f0_v, f1_v]
    msg_v = f4_v

    def issueB(c, b, bb):
        base = (wid * NCHUNK + c) * CH
        pltpu.sync_copy(sidx_hbm.at[pl.ds(base, CH)], sB[b])
        pltpu.sync_copy(didx_hbm.at[pl.ds(base, CH)], dB[b])
        for g in range(CH // 16):
            sv = sB[b][pl.ds(g * 16, 16)]
            pB[b][pl.ds(g * 16, 16)] = sv + bb * N_NODE
        pltpu.async_copy(vrelp_hbm.at[pB[b]], vB[b], s0[b])

    def computeB(c, b, bb):
        pltpu.make_async_copy(vrelp_hbm.at[pB[b]], vB[b], s0[b]).wait()
        for g in range(CH // 16):
            rows16 = iot + g * 16
            aidx = (c * CH + g * 16 + iot) * 4 + (bb >> 1)
            av16 = plsc.load_gather(a_sp, [aidx])
            for j in range(16):
                jv = jnp.full((16,), j, jnp.int32)
                val = plsc.load_gather(vB[b], [rows16, jv]) * av16
                plsc.store_scatter(msg_v, [rows16, jv], val)
        pltpu.sync_copy(msg_v, out_sp.at[dB[b]], add=True)

    def head_block(bb, carry):
        for r in range(CH):
            f5_v[r] = zero16
        for j in range(RPT // CH):
            pltpu.sync_copy(f5_v, out_sp.at[pl.ds(row0 + j * CH, CH), :])
        rem = RPT - (RPT // CH) * CH
        if rem:
            pltpu.sync_copy(f5_v.at[pl.ds(0, rem), :],
                            out_sp.at[pl.ds(row0 + (RPT // CH) * CH, rem), :])
        plsc.subcore_barrier()

        issueB(0, 0, bb)

        def pairB(p, carry2):
            c0 = p * 2
            issueB(c0 + 1, 1, bb)
            computeB(c0, 0, bb)

            @pl.when(c0 + 2 < NCHUNK)
            def _():
                issueB(c0 + 2, 0, bb)
            computeB(c0 + 1, 1, bb)
            return carry2

        lax.fori_loop(0, NCHUNK // 2, pairB, 0)
        plsc.subcore_barrier()

        @pl.when(cid == 0)
        def _():
            pltpu.sync_copy(out_sp.at[pl.ds(row0, RPT), :],
                            out0_hbm.at[pl.ds(bb * NSP + row0, RPT), :])

        @pl.when(cid == 1)
        def _():
            pltpu.sync_copy(out_sp.at[pl.ds(row0, RPT), :],
                            out1_hbm.at[pl.ds(bb * NSP + row0, RPT), :])
        plsc.subcore_barrier()
        return carry

    lax.fori_loop(0, 2 * H, head_block, 0)


def _sc_pass2(vrel_pack, s_pad, d_pad, ex, den0, den1):
    mesh = plsc.VectorSubcoreMesh(core_axis_name="c", subcore_axis_name="s",
                                  num_cores=NC, num_subcores=NS)
    fn = pl.kernel(
        _pass2_body,
        out_type=[
            jax.ShapeDtypeStruct((2 * H * NSP, 16), jnp.float32),
            jax.ShapeDtypeStruct((2 * H * NSP, 16), jnp.float32),
        ],
        mesh=mesh,
        compiler_params=pltpu.CompilerParams(
            use_tc_tiling_on_sc=False, needs_layout_passes=False),
        scratch_types=[
            pltpu.VMEM((CH,), jnp.int32),
            pltpu.VMEM((CH,), jnp.int32),
            pltpu.VMEM((CH,), jnp.int32),
            pltpu.VMEM((CH,), jnp.int32),
            pltpu.VMEM((CH,), jnp.int32),
            pltpu.VMEM((CH,), jnp.int32),
            pltpu.VMEM((CH, 16), jnp.float32),
            pltpu.VMEM((CH, 16), jnp.float32),
            pltpu.VMEM((CH, 16), jnp.float32),
            pltpu.VMEM((CH, 16), jnp.float32),
            pltpu.VMEM((CH, 16), jnp.float32),
            pltpu.VMEM((CH, 16), jnp.float32),
            pltpu.VMEM((EPT * 4,), jnp.float32),
            pltpu.VMEM_SHARED((NSP, 16), jnp.float32),
            pltpu.SemaphoreType.DMA,
            pltpu.SemaphoreType.DMA,
            pltpu.SemaphoreType.DMA,
            pltpu.SemaphoreType.DMA,
        ],
    )
    return fn(vrel_pack, s_pad, d_pad, ex, den0, den1)


def _edge_phase(q_dst, krel_src, vrel_src, s, d, n_dst):
    pad = jnp.arange(E_PAD - E_REAL, dtype=jnp.int32) % N_NODE
    s_pad = jnp.concatenate([s.astype(jnp.int32), pad])
    d_pad = jnp.concatenate([d.astype(jnp.int32), pad])
    ex, den0, den1 = _sc_pass1(q_dst, krel_src, s_pad, d_pad)
    vrel_pack = vrel_src.reshape(N_NODE, 2 * H, 16).transpose(1, 0, 2).reshape(
        2 * H * N_NODE, 16)
    o0, o1 = _sc_pass2(vrel_pack, s_pad, d_pad, ex, den0, den1)
    osum = (o0 + o1).reshape(2 * H, NSP, 16)[:, :n_dst, :]
    return osum.transpose(1, 0, 2).reshape(n_dst, C)


_SRC_EDGE = {'user': 'ui', 'item': 'iu'}
_EDGE_DEFS = (('ui', 'user', 'item'), ('iu', 'item', 'user'))


def _fold_params(params):
    folded = {}
    inv_sqrt = 1.0 / math.sqrt(float(DH))
    for l in range(2):
        for t in ('user', 'item'):
            e = _SRC_EDGE[t]
            arel = params['l%d_arel_%s' % (l, e)]
            mrel = params['l%d_mrel_%s' % (l, e)]
            prel = params['l%d_prel_%s' % (l, e)] * inv_sqrt
            Wk = params['l%d_Wk_%s' % (l, t)].reshape(C, H, DH)
            Wv = params['l%d_Wv_%s' % (l, t)].reshape(C, H, DH)
            Wk_f = jnp.einsum('chd,hde,h->che', Wk, arel, prel).reshape(C, C)
            Wv_f = jnp.einsum('chd,hde->che', Wv, mrel).reshape(C, C)
            Wq = params['l%d_Wq_%s' % (l, t)]
            folded['Wqkv_%d_%s' % (l, t)] = jnp.concatenate([Wq, Wk_f, Wv_f], axis=1)
    return folded


def kernel(x_user, x_item, edge_index_user_item, edge_index_item_user, params):
    folded = _fold_params(params)
    h = {'user': _proj_relu(x_user, params['in_W_user'], params['in_b_user']),
         'item': _proj_relu(x_item, params['in_W_item'], params['in_b_item'])}
    ei = {'ui': (edge_index_user_item[0], edge_index_user_item[1]),
          'iu': (edge_index_item_user[0], edge_index_item_user[1])}
    for l in range(2):
        q, krel, vrel = {}, {}, {}
        for t in h:
            y = _proj(h[t], folded['Wqkv_%d_%s' % (l, t)])
            q[t] = y[:, :C]
            krel[t] = y[:, C:2 * C]
            vrel[t] = y[:, 2 * C:]
        out = {}
        for e, src, dst in _EDGE_DEFS:
            s, d = ei[e]
            out[dst] = _edge_phase(q[dst], krel[src], vrel[src], s, d, N_NODE)
        h_new = {}
        for t in h:
            beta = jax.nn.sigmoid(params['l%d_skip_%s' % (l, t)])
            hcoef = (1.0 - beta) + (1.0 if l > 0 else 0.0)
            h_new[t] = _out_stage(out[t], h[t],
                                  params['l%d_Wa_%s' % (l, t)],
                                  params['l%d_ba_%s' % (l, t)], beta, hcoef)
        h = h_new
    return (h['user'], h['item'])


# speedup vs baseline: 1.9116x; 1.5342x over previous
import functools
import math

import jax
import jax.numpy as jnp
import numpy as np
from jax import lax
from jax.experimental import pallas as pl
from jax.experimental.pallas import tpu as pltpu
from jax.experimental.pallas import tpu_sc as plsc

N_NODE = 50000
C = 128
H = 4
DH = 32
BN = 2000

NC = 2
NS = 16
NW = NC * NS
CH = 128
NCHUNK = 74
EPT = NCHUNK * CH
E_PAD = NW * EPT
E_REAL = 300000
NSP = 50048
RPT = NSP // NS


def _proj_relu_body(x_ref, w_ref, b_ref, o_ref):
    y = jnp.dot(x_ref[...], w_ref[...], preferred_element_type=jnp.float32)
    o_ref[...] = jax.nn.relu(y + b_ref[...])


def _proj_relu(x, w, b):
    n = x.shape[0]
    grid = (n // BN,)
    return pl.pallas_call(
        _proj_relu_body,
        grid=grid,
        in_specs=[
            pl.BlockSpec((BN, x.shape[1]), lambda i: (i, 0)),
            pl.BlockSpec((x.shape[1], w.shape[1]), lambda i: (0, 0)),
            pl.BlockSpec((1, w.shape[1]), lambda i: (0, 0)),
        ],
        out_specs=pl.BlockSpec((BN, w.shape[1]), lambda i: (i, 0)),
        out_shape=jax.ShapeDtypeStruct((n, w.shape[1]), jnp.float32),
    )(x, w, b.reshape(1, -1))


def _proj_body(x_ref, w_ref, o_ref):
    o_ref[...] = jnp.dot(x_ref[...], w_ref[...], preferred_element_type=jnp.float32)


def _proj(x, w):
    n = x.shape[0]
    grid = (n // BN,)
    return pl.pallas_call(
        _proj_body,
        grid=grid,
        in_specs=[
            pl.BlockSpec((BN, x.shape[1]), lambda i: (i, 0)),
            pl.BlockSpec((x.shape[1], w.shape[1]), lambda i: (0, 0)),
        ],
        out_specs=pl.BlockSpec((BN, w.shape[1]), lambda i: (i, 0)),
        out_shape=jax.ShapeDtypeStruct((n, w.shape[1]), jnp.float32),
    )(x, w)


def _out_stage_body(c_ref, m_ref, h_ref, w_ref, b_ref, o_ref):
    o = jax.nn.gelu(m_ref[...])
    o = jnp.dot(o, w_ref[...], preferred_element_type=jnp.float32) + b_ref[...]
    o_ref[...] = c_ref[0] * o + c_ref[1] * h_ref[...]


def _out_stage(m, h, w, b, beta, hcoef):
    n = m.shape[0]
    grid = (n // BN,)
    coef = jnp.stack([beta, hcoef]).astype(jnp.float32)
    return pl.pallas_call(
        _out_stage_body,
        grid=grid,
        in_specs=[
            pl.BlockSpec(memory_space=pltpu.SMEM),
            pl.BlockSpec((BN, C), lambda i: (i, 0)),
            pl.BlockSpec((BN, C), lambda i: (i, 0)),
            pl.BlockSpec((C, C), lambda i: (0, 0)),
            pl.BlockSpec((1, C), lambda i: (0, 0)),
        ],
        out_specs=pl.BlockSpec((BN, C), lambda i: (i, 0)),
        out_shape=jax.ShapeDtypeStruct((n, C), jnp.float32),
    )(coef, m, h, w, b.reshape(1, -1))


def _pass1_body(q_hbm, krel_hbm, sidx_hbm, didx_hbm,
                ex_hbm, den0_hbm, den1_hbm,
                s0_v, s1_v, d0_v, d1_v, ke0_v, ke1_v, qe0_v, qe1_v,
                ex_v, den_sp, semk0, semq0, semk1, semq1):
    cid = lax.axis_index("c")
    sid = lax.axis_index("s")
    wid = cid * NS + sid
    iot = lax.iota(jnp.int32, 16)
    zero16 = jnp.zeros((16,), jnp.float32)
    sbuf = [s0_v, s1_v]
    dbuf = [d0_v, d1_v]
    kbuf = [ke0_v, ke1_v]
    qbuf = [qe0_v, qe1_v]
    ksem = [semk0, semk1]
    qsem = [semq0, semq1]

    for r in range(CH):
        ex_v[r] = zero16
    row0 = sid * RPT
    for j in range(RPT // CH):
        pltpu.sync_copy(ex_v, den_sp.at[pl.ds(row0 + j * CH, CH), :])
    rem = RPT - (RPT // CH) * CH
    if rem:
        pltpu.sync_copy(ex_v.at[pl.ds(0, rem), :],
                        den_sp.at[pl.ds(row0 + (RPT // CH) * CH, rem), :])
    plsc.subcore_barrier()

    def issue(c, b):
        base = (wid * NCHUNK + c) * CH
        pltpu.sync_copy(sidx_hbm.at[pl.ds(base, CH)], sbuf[b])
        pltpu.sync_copy(didx_hbm.at[pl.ds(base, CH)], dbuf[b])
        pltpu.async_copy(krel_hbm.at[sbuf[b]], kbuf[b], ksem[b])
        pltpu.async_copy(q_hbm.at[dbuf[b]], qbuf[b], qsem[b])

    def compute(c, b):
        pltpu.make_async_copy(krel_hbm.at[sbuf[b]], kbuf[b], ksem[b]).wait()
        pltpu.make_async_copy(q_hbm.at[dbuf[b]], qbuf[b], qsem[b]).wait()
        base = (wid * NCHUNK + c) * CH
        ke, qe = kbuf[b], qbuf[b]
        lane4 = iot < 4

        def edge_body(r, carry):
            sums = []
            for h in range(H):
                p = (ke[r, pl.ds(h * DH, 16)] * qe[r, pl.ds(h * DH, 16)]
                     + ke[r, pl.ds(h * DH + 16, 16)] * qe[r, pl.ds(h * DH + 16, 16)])
                sums.append(jnp.sum(p))
            row = jnp.where(iot == 0, sums[0], 0.0)
            for h in range(1, H):
                row = jnp.where(iot == h, sums[h], row)
            valid = lane4 & (base + r < E_REAL)
            ex_v[r] = jnp.where(valid, jnp.exp(row), 0.0)
            return carry

        lax.fori_loop(0, CH, edge_body, 0, unroll=4)
        pltpu.sync_copy(ex_v, ex_hbm.at[pl.ds(base, CH), :])
        pltpu.sync_copy(ex_v, den_sp.at[dbuf[b]], add=True)

    issue(0, 0)

    def pair_body(p, carry):
        c0 = p * 2
        issue(c0 + 1, 1)
        compute(c0, 0)

        @pl.when(c0 + 2 < NCHUNK)
        def _():
            issue(c0 + 2, 0)
        compute(c0 + 1, 1)
        return carry

    lax.fori_loop(0, NCHUNK // 2, pair_body, 0)
    plsc.subcore_barrier()

    @pl.when(cid == 0)
    def _():
        pltpu.sync_copy(den_sp.at[pl.ds(row0, RPT), :],
                        den0_hbm.at[pl.ds(row0, RPT), :])

    @pl.when(cid == 1)
    def _():
        pltpu.sync_copy(den_sp.at[pl.ds(row0, RPT), :],
                        den1_hbm.at[pl.ds(row0, RPT), :])


def _sc_pass1(q_dst, krel_src, s_pad, d_pad):
    mesh = plsc.VectorSubcoreMesh(core_axis_name="c", subcore_axis_name="s",
                                  num_cores=NC, num_subcores=NS)
    fn = pl.kernel(
        _pass1_body,
        out_type=[
            jax.ShapeDtypeStruct((E_PAD, 16), jnp.float32),
            jax.ShapeDtypeStruct((NSP, 16), jnp.float32),
            jax.ShapeDtypeStruct((NSP, 16), jnp.float32),
        ],
        mesh=mesh,
        compiler_params=pltpu.CompilerParams(
            use_tc_tiling_on_sc=False, needs_layout_passes=False),
        scratch_types=[
            pltpu.VMEM((CH,), jnp.int32),
            pltpu.VMEM((CH,), jnp.int32),
            pltpu.VMEM((CH,), jnp.int32),
            pltpu.VMEM((CH,), jnp.int32),
            pltpu.VMEM((CH, C), jnp.float32),
            pltpu.VMEM((CH, C), jnp.float32),
            pltpu.VMEM((CH, C), jnp.float32),
            pltpu.VMEM((CH, C), jnp.float32),
            pltpu.VMEM((CH, 16), jnp.float32),
            pltpu.VMEM_SHARED((NSP, 16), jnp.float32),
            pltpu.SemaphoreType.DMA,
            pltpu.SemaphoreType.DMA,
            pltpu.SemaphoreType.DMA,
            pltpu.SemaphoreType.DMA,
        ],
    )
    return fn(q_dst, krel_src, s_pad, d_pad)


def _pass2_body(vrelp_hbm, sidx_hbm, didx_hbm, ex_hbm, den0_hbm, den1_hbm,
                out0_hbm, out1_hbm,
                ia0_v, ia1_v, ib0_v, ib1_v, ic0_v, ic1_v,
                f0_v, f1_v, f2_v, f3_v, f4_v, f5_v,
                a_sp, out_sp, sm0, sm1, sm2, sm3):
    cid = lax.axis_index("c")
    sid = lax.axis_index("s")
    wid = cid * NS + sid
    iot = lax.iota(jnp.int32, 16)
    zero16 = jnp.zeros((16,), jnp.float32)
    row0 = sid * RPT

    dA = [ia0_v, ia1_v]
    g0 = [f0_v, f1_v]
    g1 = [f2_v, f3_v]
    exb = [f4_v, f5_v]
    s0 = [sm0, sm1]
    s1 = [sm2, sm3]

    def issueA(c, b):
        base = (wid * NCHUNK + c) * CH
        pltpu.sync_copy(didx_hbm.at[pl.ds(base, CH)], dA[b])
        pltpu.async_copy(den0_hbm.at[dA[b]], g0[b], s0[b])
        pltpu.async_copy(den1_hbm.at[dA[b]], g1[b], s1[b])
        pltpu.sync_copy(ex_hbm.at[pl.ds(base, CH), :], exb[b])

    lane4 = iot < 4

    def computeA(c, b):
        pltpu.make_async_copy(den0_hbm.at[dA[b]], g0[b], s0[b]).wait()
        pltpu.make_async_copy(den1_hbm.at[dA[b]], g1[b], s1[b]).wait()

        def edge_bodyA(r, carry):
            den = g0[b][r] + g1[b][r]
            a_row = exb[b][r] / (den + 1e-16)
            plsc.store_scatter(a_sp, [(c * CH + r) * 4 + iot], a_row, mask=lane4)
            return carry

        lax.fori_loop(0, CH, edge_bodyA, 0, unroll=4)

    issueA(0, 0)

    def pairA(p, carry):
        c0 = p * 2
        issueA(c0 + 1, 1)
        computeA(c0, 0)

        @pl.when(c0 + 2 < NCHUNK)
        def _():
            issueA(c0 + 2, 0)
        computeA(c0 + 1, 1)
        return carry

    lax.fori_loop(0, NCHUNK // 2, pairA, 0)

    sB = [ia0_v, ia1_v]
    dB = [ib0_v, ib1_v]
    pB = [ic0_v, ic1_v]
    vB = [f0_v, f1_v]
    msg_v = f4_v

    def issueB(c, b, bb):
        base = (wid * NCHUNK + c) * CH
        pltpu.sync_copy(sidx_hbm.at[pl.ds(base, CH)], sB[b])
        pltpu.sync_copy(didx_hbm.at[pl.ds(base, CH)], dB[b])
        for g in range(CH // 16):
            sv = sB[b][pl.ds(g * 16, 16)]
            pB[b][pl.ds(g * 16, 16)] = sv + bb * N_NODE
        pltpu.async_copy(vrelp_hbm.at[pB[b]], vB[b], s0[b])

    def computeB(c, b, bb):
        pltpu.make_async_copy(vrelp_hbm.at[pB[b]], vB[b], s0[b]).wait()
        h_of_bb = bb >> 1

        def edge_bodyB(r, carry):
            aaddr = jnp.full((16,), (c * CH + r) * 4 + h_of_bb, jnp.int32)
            av = plsc.load_gather(a_sp, [aaddr])
            msg_v[r] = vB[b][r] * av
            return carry

        lax.fori_loop(0, CH, edge_bodyB, 0, unroll=4)
        pltpu.sync_copy(msg_v, out_sp.at[dB[b]], add=True)

    def head_block(bb, carry):
        for r in range(CH):
            f5_v[r] = zero16
        for j in range(RPT // CH):
            pltpu.sync_copy(f5_v, out_sp.at[pl.ds(row0 + j * CH, CH), :])
        rem = RPT - (RPT // CH) * CH
        if rem:
            pltpu.sync_copy(f5_v.at[pl.ds(0, rem), :],
                            out_sp.at[pl.ds(row0 + (RPT // CH) * CH, rem), :])
        plsc.subcore_barrier()

        issueB(0, 0, bb)

        def pairB(p, carry2):
            c0 = p * 2
            issueB(c0 + 1, 1, bb)
            computeB(c0, 0, bb)

            @pl.when(c0 + 2 < NCHUNK)
            def _():
                issueB(c0 + 2, 0, bb)
            computeB(c0 + 1, 1, bb)
            return carry2

        lax.fori_loop(0, NCHUNK // 2, pairB, 0)
        plsc.subcore_barrier()

        @pl.when(cid == 0)
        def _():
            pltpu.sync_copy(out_sp.at[pl.ds(row0, RPT), :],
                            out0_hbm.at[pl.ds(bb * NSP + row0, RPT), :])

        @pl.when(cid == 1)
        def _():
            pltpu.sync_copy(out_sp.at[pl.ds(row0, RPT), :],
                            out1_hbm.at[pl.ds(bb * NSP + row0, RPT), :])
        plsc.subcore_barrier()
        return carry

    lax.fori_loop(0, 2 * H, head_block, 0)


def _sc_pass2(vrel_pack, s_pad, d_pad, ex, den0, den1):
    mesh = plsc.VectorSubcoreMesh(core_axis_name="c", subcore_axis_name="s",
                                  num_cores=NC, num_subcores=NS)
    fn = pl.kernel(
        _pass2_body,
        out_type=[
            jax.ShapeDtypeStruct((2 * H * NSP, 16), jnp.float32),
            jax.ShapeDtypeStruct((2 * H * NSP, 16), jnp.float32),
        ],
        mesh=mesh,
        compiler_params=pltpu.CompilerParams(
            use_tc_tiling_on_sc=False, needs_layout_passes=False),
        scratch_types=[
            pltpu.VMEM((CH,), jnp.int32),
            pltpu.VMEM((CH,), jnp.int32),
            pltpu.VMEM((CH,), jnp.int32),
            pltpu.VMEM((CH,), jnp.int32),
            pltpu.VMEM((CH,), jnp.int32),
            pltpu.VMEM((CH,), jnp.int32),
            pltpu.VMEM((CH, 16), jnp.float32),
            pltpu.VMEM((CH, 16), jnp.float32),
            pltpu.VMEM((CH, 16), jnp.float32),
            pltpu.VMEM((CH, 16), jnp.float32),
            pltpu.VMEM((CH, 16), jnp.float32),
            pltpu.VMEM((CH, 16), jnp.float32),
            pltpu.VMEM((EPT * 4,), jnp.float32),
            pltpu.VMEM_SHARED((NSP, 16), jnp.float32),
            pltpu.SemaphoreType.DMA,
            pltpu.SemaphoreType.DMA,
            pltpu.SemaphoreType.DMA,
            pltpu.SemaphoreType.DMA,
        ],
    )
    return fn(vrel_pack, s_pad, d_pad, ex, den0, den1)


def _edge_phase(q_dst, krel_src, vrel_src, s, d, n_dst):
    pad = jnp.arange(E_PAD - E_REAL, dtype=jnp.int32) % N_NODE
    s_pad = jnp.concatenate([s.astype(jnp.int32), pad])
    d_pad = jnp.concatenate([d.astype(jnp.int32), pad])
    ex, den0, den1 = _sc_pass1(q_dst, krel_src, s_pad, d_pad)
    vrel_pack = vrel_src.reshape(N_NODE, 2 * H, 16).transpose(1, 0, 2).reshape(
        2 * H * N_NODE, 16)
    o0, o1 = _sc_pass2(vrel_pack, s_pad, d_pad, ex, den0, den1)
    osum = (o0 + o1).reshape(2 * H, NSP, 16)[:, :n_dst, :]
    return osum.transpose(1, 0, 2).reshape(n_dst, C)


_SRC_EDGE = {'user': 'ui', 'item': 'iu'}
_EDGE_DEFS = (('ui', 'user', 'item'), ('iu', 'item', 'user'))


def _fold_params(params):
    folded = {}
    inv_sqrt = 1.0 / math.sqrt(float(DH))
    for l in range(2):
        for t in ('user', 'item'):
            e = _SRC_EDGE[t]
            arel = params['l%d_arel_%s' % (l, e)]
            mrel = params['l%d_mrel_%s' % (l, e)]
            prel = params['l%d_prel_%s' % (l, e)] * inv_sqrt
            Wk = params['l%d_Wk_%s' % (l, t)].reshape(C, H, DH)
            Wv = params['l%d_Wv_%s' % (l, t)].reshape(C, H, DH)
            Wk_f = jnp.einsum('chd,hde,h->che', Wk, arel, prel).reshape(C, C)
            Wv_f = jnp.einsum('chd,hde->che', Wv, mrel).reshape(C, C)
            Wq = params['l%d_Wq_%s' % (l, t)]
            folded['Wqkv_%d_%s' % (l, t)] = jnp.concatenate([Wq, Wk_f, Wv_f], axis=1)
    return folded


def kernel(x_user, x_item, edge_index_user_item, edge_index_item_user, params):
    folded = _fold_params(params)
    h = {'user': _proj_relu(x_user, params['in_W_user'], params['in_b_user']),
         'item': _proj_relu(x_item, params['in_W_item'], params['in_b_item'])}
    ei = {'ui': (edge_index_user_item[0], edge_index_user_item[1]),
          'iu': (edge_index_item_user[0], edge_index_item_user[1])}
    for l in range(2):
        q, krel, vrel = {}, {}, {}
        for t in h:
            y = _proj(h[t], folded['Wqkv_%d_%s' % (l, t)])
            q[t] = y[:, :C]
            krel[t] = y[:, C:2 * C]
            vrel[t] = y[:, 2 * C:]
        out = {}
        for e, src, dst in _EDGE_DEFS:
            s, d = ei[e]
            out[dst] = _edge_phase(q[dst], krel[src], vrel[src], s, d, N_NODE)
        h_new = {}
        for t in h:
            beta = jax.nn.sigmoid(params['l%d_skip_%s' % (l, t)])
            hcoef = (1.0 - beta) + (1.0 if l > 0 else 0.0)
            h_new[t] = _out_stage(out[t], h[t],
                                  params['l%d_Wa_%s' % (l, t)],
                                  params['l%d_ba_%s' % (l, t)], beta, hcoef)
        h = h_new
    return (h['user'], h['item'])


# speedup vs baseline: 2.5961x; 1.3581x over previous
import functools
import math

import jax
import jax.numpy as jnp
import numpy as np
from jax import lax
from jax.experimental import pallas as pl
from jax.experimental.pallas import tpu as pltpu
from jax.experimental.pallas import tpu_sc as plsc

N_NODE = 50000
C = 128
H = 4
DH = 32
BN = 2000

NC = 2
NS = 16
NW = NC * NS
CH = 128
NCHUNK = 74
EPT = NCHUNK * CH
E_PAD = NW * EPT
E_REAL = 300000
NSP = 50048
RPT = NSP // NS


def _proj_relu_body(x_ref, w_ref, b_ref, o_ref):
    y = jnp.dot(x_ref[...], w_ref[...], preferred_element_type=jnp.float32)
    o_ref[...] = jax.nn.relu(y + b_ref[...])


def _proj_relu(x, w, b):
    n = x.shape[0]
    grid = (n // BN,)
    return pl.pallas_call(
        _proj_relu_body,
        grid=grid,
        in_specs=[
            pl.BlockSpec((BN, x.shape[1]), lambda i: (i, 0)),
            pl.BlockSpec((x.shape[1], w.shape[1]), lambda i: (0, 0)),
            pl.BlockSpec((1, w.shape[1]), lambda i: (0, 0)),
        ],
        out_specs=pl.BlockSpec((BN, w.shape[1]), lambda i: (i, 0)),
        out_shape=jax.ShapeDtypeStruct((n, w.shape[1]), jnp.float32),
    )(x, w, b.reshape(1, -1))


def _proj_body(x_ref, w_ref, o_ref):
    o_ref[...] = jnp.dot(x_ref[...], w_ref[...], preferred_element_type=jnp.float32)


def _proj(x, w):
    n = x.shape[0]
    grid = (n // BN,)
    return pl.pallas_call(
        _proj_body,
        grid=grid,
        in_specs=[
            pl.BlockSpec((BN, x.shape[1]), lambda i: (i, 0)),
            pl.BlockSpec((x.shape[1], w.shape[1]), lambda i: (0, 0)),
        ],
        out_specs=pl.BlockSpec((BN, w.shape[1]), lambda i: (i, 0)),
        out_shape=jax.ShapeDtypeStruct((n, w.shape[1]), jnp.float32),
    )(x, w)


def _out_stage_body(c_ref, m_ref, h_ref, w_ref, b_ref, o_ref):
    o = jax.nn.gelu(m_ref[...])
    o = jnp.dot(o, w_ref[...], preferred_element_type=jnp.float32) + b_ref[...]
    o_ref[...] = c_ref[0] * o + c_ref[1] * h_ref[...]


def _out_stage(m, h, w, b, beta, hcoef):
    n = m.shape[0]
    grid = (n // BN,)
    coef = jnp.stack([beta, hcoef]).astype(jnp.float32)
    return pl.pallas_call(
        _out_stage_body,
        grid=grid,
        in_specs=[
            pl.BlockSpec(memory_space=pltpu.SMEM),
            pl.BlockSpec((BN, C), lambda i: (i, 0)),
            pl.BlockSpec((BN, C), lambda i: (i, 0)),
            pl.BlockSpec((C, C), lambda i: (0, 0)),
            pl.BlockSpec((1, C), lambda i: (0, 0)),
        ],
        out_specs=pl.BlockSpec((BN, C), lambda i: (i, 0)),
        out_shape=jax.ShapeDtypeStruct((n, C), jnp.float32),
    )(coef, m, h, w, b.reshape(1, -1))


def _pass1_body(q_hbm, krel_hbm, sidx_hbm, didx_hbm,
                ex_hbm, den0_hbm, den1_hbm,
                s0_v, s1_v, d0_v, d1_v, ke0_v, ke1_v, qe0_v, qe1_v,
                ex_v, den_sp, semk0, semq0, semk1, semq1):
    cid = lax.axis_index("c")
    sid = lax.axis_index("s")
    wid = cid * NS + sid
    iot = lax.iota(jnp.int32, 16)
    zero16 = jnp.zeros((16,), jnp.float32)
    sbuf = [s0_v, s1_v]
    dbuf = [d0_v, d1_v]
    kbuf = [ke0_v, ke1_v]
    qbuf = [qe0_v, qe1_v]
    ksem = [semk0, semk1]
    qsem = [semq0, semq1]

    for r in range(CH):
        ex_v[r] = zero16
    row0 = sid * RPT
    for j in range(RPT // CH):
        pltpu.sync_copy(ex_v, den_sp.at[pl.ds(row0 + j * CH, CH), :])
    rem = RPT - (RPT // CH) * CH
    if rem:
        pltpu.sync_copy(ex_v.at[pl.ds(0, rem), :],
                        den_sp.at[pl.ds(row0 + (RPT // CH) * CH, rem), :])
    plsc.subcore_barrier()

    def issue(c, b):
        base = (wid * NCHUNK + c) * CH
        pltpu.sync_copy(sidx_hbm.at[pl.ds(base, CH)], sbuf[b])
        pltpu.sync_copy(didx_hbm.at[pl.ds(base, CH)], dbuf[b])
        pltpu.async_copy(krel_hbm.at[sbuf[b]], kbuf[b], ksem[b])
        pltpu.async_copy(q_hbm.at[dbuf[b]], qbuf[b], qsem[b])

    def compute(c, b):
        pltpu.make_async_copy(krel_hbm.at[sbuf[b]], kbuf[b], ksem[b]).wait()
        pltpu.make_async_copy(q_hbm.at[dbuf[b]], qbuf[b], qsem[b]).wait()
        base = (wid * NCHUNK + c) * CH
        ke, qe = kbuf[b], qbuf[b]
        lane4 = iot < 4

        def edge_body(r, carry):
            sums = []
            for h in range(H):
                p = (ke[r, pl.ds(h * DH, 16)] * qe[r, pl.ds(h * DH, 16)]
                     + ke[r, pl.ds(h * DH + 16, 16)] * qe[r, pl.ds(h * DH + 16, 16)])
                sums.append(jnp.sum(p))
            row = jnp.where(iot == 0, sums[0], 0.0)
            for h in range(1, H):
                row = jnp.where(iot == h, sums[h], row)
            valid = lane4 & (base + r < E_REAL)
            ex_v[r] = jnp.where(valid, jnp.exp(row), 0.0)
            return carry

        lax.fori_loop(0, CH, edge_body, 0, unroll=4)
        pltpu.sync_copy(ex_v, ex_hbm.at[pl.ds(base, CH), :])
        pltpu.sync_copy(ex_v, den_sp.at[dbuf[b]], add=True)

    issue(0, 0)

    def pair_body(p, carry):
        c0 = p * 2
        issue(c0 + 1, 1)
        compute(c0, 0)

        @pl.when(c0 + 2 < NCHUNK)
        def _():
            issue(c0 + 2, 0)
        compute(c0 + 1, 1)
        return carry

    lax.fori_loop(0, NCHUNK // 2, pair_body, 0)
    plsc.subcore_barrier()

    @pl.when(cid == 0)
    def _():
        pltpu.sync_copy(den_sp.at[pl.ds(row0, RPT), :],
                        den0_hbm.at[pl.ds(row0, RPT), :])

    @pl.when(cid == 1)
    def _():
        pltpu.sync_copy(den_sp.at[pl.ds(row0, RPT), :],
                        den1_hbm.at[pl.ds(row0, RPT), :])


def _sc_pass1(q_dst, krel_src, s_pad, d_pad):
    mesh = plsc.VectorSubcoreMesh(core_axis_name="c", subcore_axis_name="s",
                                  num_cores=NC, num_subcores=NS)
    fn = pl.kernel(
        _pass1_body,
        out_type=[
            jax.ShapeDtypeStruct((E_PAD, 16), jnp.float32),
            jax.ShapeDtypeStruct((NSP, 16), jnp.float32),
            jax.ShapeDtypeStruct((NSP, 16), jnp.float32),
        ],
        mesh=mesh,
        compiler_params=pltpu.CompilerParams(
            use_tc_tiling_on_sc=False, needs_layout_passes=False),
        scratch_types=[
            pltpu.VMEM((CH,), jnp.int32),
            pltpu.VMEM((CH,), jnp.int32),
            pltpu.VMEM((CH,), jnp.int32),
            pltpu.VMEM((CH,), jnp.int32),
            pltpu.VMEM((CH, C), jnp.float32),
            pltpu.VMEM((CH, C), jnp.float32),
            pltpu.VMEM((CH, C), jnp.float32),
            pltpu.VMEM((CH, C), jnp.float32),
            pltpu.VMEM((CH, 16), jnp.float32),
            pltpu.VMEM_SHARED((NSP, 16), jnp.float32),
            pltpu.SemaphoreType.DMA,
            pltpu.SemaphoreType.DMA,
            pltpu.SemaphoreType.DMA,
            pltpu.SemaphoreType.DMA,
        ],
    )
    return fn(q_dst, krel_src, s_pad, d_pad)


def _pass2_body(vrelp_hbm, sidx_hbm, didx_hbm, ex_hbm, den0_hbm, den1_hbm,
                out0_hbm, out1_hbm,
                sall_v, dall_v, ic0_v, ic1_v,
                f0_v, f1_v, f2_v, f3_v, f4_v, f5_v,
                a_sp, out_sp, sm0, sm1, sm2, sm3):
    cid = lax.axis_index("c")
    sid = lax.axis_index("s")
    wid = cid * NS + sid
    iot = lax.iota(jnp.int32, 16)
    zero16 = jnp.zeros((16,), jnp.float32)
    row0 = sid * RPT

    pltpu.sync_copy(sidx_hbm.at[pl.ds(wid * EPT, EPT)], sall_v)
    pltpu.sync_copy(didx_hbm.at[pl.ds(wid * EPT, EPT)], dall_v)

    g0 = [f0_v, f1_v]
    g1 = [f2_v, f3_v]
    exb = [f4_v, f5_v]
    s0 = [sm0, sm1]
    s1 = [sm2, sm3]
    lane4 = iot < 4

    def issueA(c, b):
        base = (wid * NCHUNK + c) * CH
        idx = dall_v.at[pl.ds(c * CH, CH)]
        pltpu.async_copy(den0_hbm.at[idx], g0[b], s0[b])
        pltpu.async_copy(den1_hbm.at[idx], g1[b], s1[b])
        pltpu.sync_copy(ex_hbm.at[pl.ds(base, CH), :], exb[b])

    def computeA(c, b):
        idx = dall_v.at[pl.ds(c * CH, CH)]
        pltpu.make_async_copy(den0_hbm.at[idx], g0[b], s0[b]).wait()
        pltpu.make_async_copy(den1_hbm.at[idx], g1[b], s1[b]).wait()

        def edge_bodyA(r, carry):
            den = g0[b][r] + g1[b][r]
            a_row = exb[b][r] / (den + 1e-16)
            plsc.store_scatter(a_sp, [(c * CH + r) * 4 + iot], a_row, mask=lane4)
            return carry

        lax.fori_loop(0, CH, edge_bodyA, 0, unroll=4)

    issueA(0, 0)

    def pairA(p, carry):
        c0 = p * 2
        issueA(c0 + 1, 1)
        computeA(c0, 0)

        @pl.when(c0 + 2 < NCHUNK)
        def _():
            issueA(c0 + 2, 0)
        computeA(c0 + 1, 1)
        return carry

    lax.fori_loop(0, NCHUNK // 2, pairA, 0)

    pB = [ic0_v, ic1_v]
    vB = [f0_v, f1_v]
    msg = [f4_v, f5_v]
    zbuf = f2_v
    for r in range(CH):
        zbuf[r] = zero16

    def issueB(c, b, bb):
        for g in range(CH // 16):
            sv = sall_v[pl.ds(c * CH + g * 16, 16)]
            pB[b][pl.ds(g * 16, 16)] = sv + bb * N_NODE
        pltpu.async_copy(vrelp_hbm.at[pB[b]], vB[b], s0[b])

    def computeB(c, b, bb, first):
        pltpu.make_async_copy(vrelp_hbm.at[pB[b]], vB[b], s0[b]).wait()
        didx = dall_v.at[pl.ds(c * CH, CH)]
        if not first:
            pltpu.make_async_copy(msg[b], out_sp.at[didx], s1[b]).wait()
        h_of_bb = bb >> 1

        def edge_bodyB(r, carry):
            aaddr = jnp.full((16,), (c * CH + r) * 4 + h_of_bb, jnp.int32)
            av = plsc.load_gather(a_sp, [aaddr])
            msg[b][r] = vB[b][r] * av
            return carry

        lax.fori_loop(0, CH, edge_bodyB, 0, unroll=8)
        pltpu.async_copy(msg[b], out_sp.at[didx], s1[b], add=True)

    def head_block(bb, carry):
        for j in range(RPT // CH):
            pltpu.sync_copy(zbuf, out_sp.at[pl.ds(row0 + j * CH, CH), :])
        rem = RPT - (RPT // CH) * CH
        if rem:
            pltpu.sync_copy(zbuf.at[pl.ds(0, rem), :],
                            out_sp.at[pl.ds(row0 + (RPT // CH) * CH, rem), :])
        plsc.subcore_barrier()

        issueB(0, 0, bb)
        issueB(1, 1, bb)
        computeB(0, 0, bb, True)
        issueB(2, 0, bb)
        computeB(1, 1, bb, True)

        def pairB(p, carry2):
            c0 = p * 2
            issueB(c0 + 3, 1, bb)
            computeB(c0 + 2, 0, bb, False)

            @pl.when(c0 + 4 < NCHUNK)
            def _():
                issueB(c0 + 4, 0, bb)
            computeB(c0 + 3, 1, bb, False)
            return carry2

        lax.fori_loop(0, NCHUNK // 2 - 1, pairB, 0)
        dd0 = dall_v.at[pl.ds((NCHUNK - 2) * CH, CH)]
        dd1 = dall_v.at[pl.ds((NCHUNK - 1) * CH, CH)]
        pltpu.make_async_copy(msg[0], out_sp.at[dd0], s1[0]).wait()
        pltpu.make_async_copy(msg[1], out_sp.at[dd1], s1[1]).wait()
        plsc.subcore_barrier()

        @pl.when(cid == 0)
        def _():
            pltpu.sync_copy(out_sp.at[pl.ds(row0, RPT), :],
                            out0_hbm.at[pl.ds(bb * NSP + row0, RPT), :])

        @pl.when(cid == 1)
        def _():
            pltpu.sync_copy(out_sp.at[pl.ds(row0, RPT), :],
                            out1_hbm.at[pl.ds(bb * NSP + row0, RPT), :])
        plsc.subcore_barrier()
        return carry

    lax.fori_loop(0, 2 * H, head_block, 0)


def _sc_pass2(vrel_pack, s_pad, d_pad, ex, den0, den1):
    mesh = plsc.VectorSubcoreMesh(core_axis_name="c", subcore_axis_name="s",
                                  num_cores=NC, num_subcores=NS)
    fn = pl.kernel(
        _pass2_body,
        out_type=[
            jax.ShapeDtypeStruct((2 * H * NSP, 16), jnp.float32),
            jax.ShapeDtypeStruct((2 * H * NSP, 16), jnp.float32),
        ],
        mesh=mesh,
        compiler_params=pltpu.CompilerParams(
            use_tc_tiling_on_sc=False, needs_layout_passes=False),
        scratch_types=[
            pltpu.VMEM((EPT,), jnp.int32),
            pltpu.VMEM((EPT,), jnp.int32),
            pltpu.VMEM((CH,), jnp.int32),
            pltpu.VMEM((CH,), jnp.int32),
            pltpu.VMEM((CH, 16), jnp.float32),
            pltpu.VMEM((CH, 16), jnp.float32),
            pltpu.VMEM((CH, 16), jnp.float32),
            pltpu.VMEM((CH, 16), jnp.float32),
            pltpu.VMEM((CH, 16), jnp.float32),
            pltpu.VMEM((CH, 16), jnp.float32),
            pltpu.VMEM((EPT * 4,), jnp.float32),
            pltpu.VMEM_SHARED((NSP, 16), jnp.float32),
            pltpu.SemaphoreType.DMA,
            pltpu.SemaphoreType.DMA,
            pltpu.SemaphoreType.DMA,
            pltpu.SemaphoreType.DMA,
        ],
    )
    return fn(vrel_pack, s_pad, d_pad, ex, den0, den1)


def _edge_phase(q_dst, krel_src, vrel_src, s, d, n_dst):
    pad = jnp.arange(E_PAD - E_REAL, dtype=jnp.int32) % N_NODE
    s_pad = jnp.concatenate([s.astype(jnp.int32), pad])
    d_pad = jnp.concatenate([d.astype(jnp.int32), pad])
    ex, den0, den1 = _sc_pass1(q_dst, krel_src, s_pad, d_pad)
    vrel_pack = vrel_src.reshape(N_NODE, 2 * H, 16).transpose(1, 0, 2).reshape(
        2 * H * N_NODE, 16)
    o0, o1 = _sc_pass2(vrel_pack, s_pad, d_pad, ex, den0, den1)
    osum = (o0 + o1).reshape(2 * H, NSP, 16)[:, :n_dst, :]
    return osum.transpose(1, 0, 2).reshape(n_dst, C)


_SRC_EDGE = {'user': 'ui', 'item': 'iu'}
_EDGE_DEFS = (('ui', 'user', 'item'), ('iu', 'item', 'user'))


def _fold_params(params):
    folded = {}
    inv_sqrt = 1.0 / math.sqrt(float(DH))
    for l in range(2):
        for t in ('user', 'item'):
            e = _SRC_EDGE[t]
            arel = params['l%d_arel_%s' % (l, e)]
            mrel = params['l%d_mrel_%s' % (l, e)]
            prel = params['l%d_prel_%s' % (l, e)] * inv_sqrt
            Wk = params['l%d_Wk_%s' % (l, t)].reshape(C, H, DH)
            Wv = params['l%d_Wv_%s' % (l, t)].reshape(C, H, DH)
            Wk_f = jnp.einsum('chd,hde,h->che', Wk, arel, prel).reshape(C, C)
            Wv_f = jnp.einsum('chd,hde->che', Wv, mrel).reshape(C, C)
            Wq = params['l%d_Wq_%s' % (l, t)]
            folded['Wqkv_%d_%s' % (l, t)] = jnp.concatenate([Wq, Wk_f, Wv_f], axis=1)
    return folded


def kernel(x_user, x_item, edge_index_user_item, edge_index_item_user, params):
    folded = _fold_params(params)
    h = {'user': _proj_relu(x_user, params['in_W_user'], params['in_b_user']),
         'item': _proj_relu(x_item, params['in_W_item'], params['in_b_item'])}
    ei = {'ui': (edge_index_user_item[0], edge_index_user_item[1]),
          'iu': (edge_index_item_user[0], edge_index_item_user[1])}
    for l in range(2):
        q, krel, vrel = {}, {}, {}
        for t in h:
            y = _proj(h[t], folded['Wqkv_%d_%s' % (l, t)])
            q[t] = y[:, :C]
            krel[t] = y[:, C:2 * C]
            vrel[t] = y[:, 2 * C:]
        out = {}
        for e, src, dst in _EDGE_DEFS:
            s, d = ei[e]
            out[dst] = _edge_phase(q[dst], krel[src], vrel[src], s, d, N_NODE)
        h_new = {}
        for t in h:
            beta = jax.nn.sigmoid(params['l%d_skip_%s' % (l, t)])
            hcoef = (1.0 - beta) + (1.0 if l > 0 else 0.0)
            h_new[t] = _out_stage(out[t], h[t],
                                  params['l%d_Wa_%s' % (l, t)],
                                  params['l%d_ba_%s' % (l, t)], beta, hcoef)
        h = h_new
    return (h['user'], h['item'])


# speedup vs baseline: 2.6086x; 1.0048x over previous
import functools
import math

import jax
import jax.numpy as jnp
import numpy as np
from jax import lax
from jax.experimental import pallas as pl
from jax.experimental.pallas import tpu as pltpu
from jax.experimental.pallas import tpu_sc as plsc

N_NODE = 50000
C = 128
H = 4
DH = 32
BN = 2000

NC = 2
NS = 16
NW = NC * NS
CH = 128
NCHUNK = 74
EPT = NCHUNK * CH
E_PAD = NW * EPT
E_REAL = 300000
NSP = 50048
RPT = NSP // NS


def _proj_relu_body(x_ref, w_ref, b_ref, o_ref):
    y = jnp.dot(x_ref[...], w_ref[...], preferred_element_type=jnp.float32)
    o_ref[...] = jax.nn.relu(y + b_ref[...])


def _proj_relu(x, w, b):
    n = x.shape[0]
    grid = (n // BN,)
    return pl.pallas_call(
        _proj_relu_body,
        grid=grid,
        in_specs=[
            pl.BlockSpec((BN, x.shape[1]), lambda i: (i, 0)),
            pl.BlockSpec((x.shape[1], w.shape[1]), lambda i: (0, 0)),
            pl.BlockSpec((1, w.shape[1]), lambda i: (0, 0)),
        ],
        out_specs=pl.BlockSpec((BN, w.shape[1]), lambda i: (i, 0)),
        out_shape=jax.ShapeDtypeStruct((n, w.shape[1]), jnp.float32),
    )(x, w, b.reshape(1, -1))


def _proj_body(x_ref, w_ref, o_ref):
    o_ref[...] = jnp.dot(x_ref[...], w_ref[...], preferred_element_type=jnp.float32)


def _proj(x, w):
    n = x.shape[0]
    grid = (n // BN,)
    return pl.pallas_call(
        _proj_body,
        grid=grid,
        in_specs=[
            pl.BlockSpec((BN, x.shape[1]), lambda i: (i, 0)),
            pl.BlockSpec((x.shape[1], w.shape[1]), lambda i: (0, 0)),
        ],
        out_specs=pl.BlockSpec((BN, w.shape[1]), lambda i: (i, 0)),
        out_shape=jax.ShapeDtypeStruct((n, w.shape[1]), jnp.float32),
    )(x, w)


def _out_stage_body(c_ref, m_ref, h_ref, w_ref, b_ref, o_ref):
    o = jax.nn.gelu(m_ref[...])
    o = jnp.dot(o, w_ref[...], preferred_element_type=jnp.float32) + b_ref[...]
    o_ref[...] = c_ref[0] * o + c_ref[1] * h_ref[...]


def _out_stage(m, h, w, b, beta, hcoef):
    n = m.shape[0]
    grid = (n // BN,)
    coef = jnp.stack([beta, hcoef]).astype(jnp.float32)
    return pl.pallas_call(
        _out_stage_body,
        grid=grid,
        in_specs=[
            pl.BlockSpec(memory_space=pltpu.SMEM),
            pl.BlockSpec((BN, C), lambda i: (i, 0)),
            pl.BlockSpec((BN, C), lambda i: (i, 0)),
            pl.BlockSpec((C, C), lambda i: (0, 0)),
            pl.BlockSpec((1, C), lambda i: (0, 0)),
        ],
        out_specs=pl.BlockSpec((BN, C), lambda i: (i, 0)),
        out_shape=jax.ShapeDtypeStruct((n, C), jnp.float32),
    )(coef, m, h, w, b.reshape(1, -1))


def _pass1_body(q_hbm, krel_hbm, sidx_hbm, didx_hbm,
                ex_hbm, den0_hbm, den1_hbm,
                s0_v, s1_v, d0_v, d1_v, ke0_v, ke1_v, qe0_v, qe1_v,
                ex0_v, ex1_v, den_sp, semk0, semq0, semk1, semq1,
                seme0, seme1, semd0, semd1):
    cid = lax.axis_index("c")
    sid = lax.axis_index("s")
    wid = cid * NS + sid
    iot = lax.iota(jnp.int32, 16)
    zero16 = jnp.zeros((16,), jnp.float32)
    sbuf = [s0_v, s1_v]
    dbuf = [d0_v, d1_v]
    kbuf = [ke0_v, ke1_v]
    qbuf = [qe0_v, qe1_v]
    ksem = [semk0, semk1]
    qsem = [semq0, semq1]
    exbuf = [ex0_v, ex1_v]
    esem = [seme0, seme1]
    dsem = [semd0, semd1]
    ex_v = ex0_v

    for r in range(CH):
        ex_v[r] = zero16
    row0 = sid * RPT
    for j in range(RPT // CH):
        pltpu.sync_copy(ex_v, den_sp.at[pl.ds(row0 + j * CH, CH), :])
    rem = RPT - (RPT // CH) * CH
    if rem:
        pltpu.sync_copy(ex_v.at[pl.ds(0, rem), :],
                        den_sp.at[pl.ds(row0 + (RPT // CH) * CH, rem), :])
    plsc.subcore_barrier()

    def issue(c, b, first):
        base = (wid * NCHUNK + c) * CH
        if not first:
            pltpu.make_async_copy(exbuf[b], ex_hbm.at[pl.ds(base, CH), :],
                                  esem[b]).wait()
            pltpu.make_async_copy(exbuf[b], den_sp.at[dbuf[b]], dsem[b]).wait()
        pltpu.sync_copy(sidx_hbm.at[pl.ds(base, CH)], sbuf[b])
        pltpu.sync_copy(didx_hbm.at[pl.ds(base, CH)], dbuf[b])
        pltpu.async_copy(krel_hbm.at[sbuf[b]], kbuf[b], ksem[b])
        pltpu.async_copy(q_hbm.at[dbuf[b]], qbuf[b], qsem[b])

    def compute(c, b):
        pltpu.make_async_copy(krel_hbm.at[sbuf[b]], kbuf[b], ksem[b]).wait()
        pltpu.make_async_copy(q_hbm.at[dbuf[b]], qbuf[b], qsem[b]).wait()
        base = (wid * NCHUNK + c) * CH
        ke, qe = kbuf[b], qbuf[b]
        ex_v = exbuf[b]
        lane4 = iot < 4

        def edge_body(r, carry):
            sums = []
            for h in range(H):
                p = (ke[r, pl.ds(h * DH, 16)] * qe[r, pl.ds(h * DH, 16)]
                     + ke[r, pl.ds(h * DH + 16, 16)] * qe[r, pl.ds(h * DH + 16, 16)])
                sums.append(jnp.sum(p))
            row = jnp.where(iot == 0, sums[0], 0.0)
            for h in range(1, H):
                row = jnp.where(iot == h, sums[h], row)
            valid = lane4 & (base + r < E_REAL)
            ex_v[r] = jnp.where(valid, jnp.exp(row), 0.0)
            return carry

        lax.fori_loop(0, CH, edge_body, 0, unroll=4)
        pltpu.async_copy(ex_v, ex_hbm.at[pl.ds(base, CH), :], esem[b])
        pltpu.async_copy(ex_v, den_sp.at[dbuf[b]], dsem[b], add=True)

    issue(0, 0, True)
    issue(1, 1, True)
    compute(0, 0)
    issue(2, 0, False)
    compute(1, 1)

    def pair_body(p, carry):
        c0 = p * 2
        issue(c0 + 3, 1, False)
        compute(c0 + 2, 0)

        @pl.when(c0 + 4 < NCHUNK)
        def _():
            issue(c0 + 4, 0, False)
        compute(c0 + 3, 1)
        return carry

    lax.fori_loop(0, NCHUNK // 2 - 1, pair_body, 0)
    for b in range(2):
        base = (wid * NCHUNK + NCHUNK - 2 + b) * CH
        pltpu.make_async_copy(exbuf[b], ex_hbm.at[pl.ds(base, CH), :],
                              esem[b]).wait()
        pltpu.make_async_copy(exbuf[b], den_sp.at[dbuf[b]], dsem[b]).wait()
    plsc.subcore_barrier()

    @pl.when(cid == 0)
    def _():
        pltpu.sync_copy(den_sp.at[pl.ds(row0, RPT), :],
                        den0_hbm.at[pl.ds(row0, RPT), :])

    @pl.when(cid == 1)
    def _():
        pltpu.sync_copy(den_sp.at[pl.ds(row0, RPT), :],
                        den1_hbm.at[pl.ds(row0, RPT), :])


def _sc_pass1(q_dst, krel_src, s_pad, d_pad):
    mesh = plsc.VectorSubcoreMesh(core_axis_name="c", subcore_axis_name="s",
                                  num_cores=NC, num_subcores=NS)
    fn = pl.kernel(
        _pass1_body,
        out_type=[
            jax.ShapeDtypeStruct((E_PAD, 16), jnp.float32),
            jax.ShapeDtypeStruct((NSP, 16), jnp.float32),
            jax.ShapeDtypeStruct((NSP, 16), jnp.float32),
        ],
        mesh=mesh,
        compiler_params=pltpu.CompilerParams(
            use_tc_tiling_on_sc=False, needs_layout_passes=False),
        scratch_types=[
            pltpu.VMEM((CH,), jnp.int32),
            pltpu.VMEM((CH,), jnp.int32),
            pltpu.VMEM((CH,), jnp.int32),
            pltpu.VMEM((CH,), jnp.int32),
            pltpu.VMEM((CH, C), jnp.float32),
            pltpu.VMEM((CH, C), jnp.float32),
            pltpu.VMEM((CH, C), jnp.float32),
            pltpu.VMEM((CH, C), jnp.float32),
            pltpu.VMEM((CH, 16), jnp.float32),
            pltpu.VMEM((CH, 16), jnp.float32),
            pltpu.VMEM_SHARED((NSP, 16), jnp.float32),
            pltpu.SemaphoreType.DMA,
            pltpu.SemaphoreType.DMA,
            pltpu.SemaphoreType.DMA,
            pltpu.SemaphoreType.DMA,
            pltpu.SemaphoreType.DMA,
            pltpu.SemaphoreType.DMA,
            pltpu.SemaphoreType.DMA,
            pltpu.SemaphoreType.DMA,
        ],
    )
    return fn(q_dst, krel_src, s_pad, d_pad)


def _pass2_body(vrelp_hbm, sidx_hbm, didx_hbm, ex_hbm, den0_hbm, den1_hbm,
                out0_hbm, out1_hbm,
                sall_v, dall_v, ic0_v, ic1_v,
                f0_v, f1_v, f2_v, f3_v, f4_v, f5_v,
                a_sp, out_sp, sm0, sm1, sm2, sm3):
    cid = lax.axis_index("c")
    sid = lax.axis_index("s")
    wid = cid * NS + sid
    iot = lax.iota(jnp.int32, 16)
    zero16 = jnp.zeros((16,), jnp.float32)
    row0 = sid * RPT

    pltpu.sync_copy(sidx_hbm.at[pl.ds(wid * EPT, EPT)], sall_v)
    pltpu.sync_copy(didx_hbm.at[pl.ds(wid * EPT, EPT)], dall_v)

    g0 = [f0_v, f1_v]
    g1 = [f2_v, f3_v]
    exb = [f4_v, f5_v]
    s0 = [sm0, sm1]
    s1 = [sm2, sm3]
    lane4 = iot < 4

    def issueA(c, b):
        base = (wid * NCHUNK + c) * CH
        idx = dall_v.at[pl.ds(c * CH, CH)]
        pltpu.async_copy(den0_hbm.at[idx], g0[b], s0[b])
        pltpu.async_copy(den1_hbm.at[idx], g1[b], s1[b])
        pltpu.sync_copy(ex_hbm.at[pl.ds(base, CH), :], exb[b])

    def computeA(c, b):
        idx = dall_v.at[pl.ds(c * CH, CH)]
        pltpu.make_async_copy(den0_hbm.at[idx], g0[b], s0[b]).wait()
        pltpu.make_async_copy(den1_hbm.at[idx], g1[b], s1[b]).wait()

        def edge_bodyA(r, carry):
            den = g0[b][r] + g1[b][r]
            a_row = exb[b][r] / (den + 1e-16)
            plsc.store_scatter(a_sp, [(c * CH + r) * 4 + iot], a_row, mask=lane4)
            return carry

        lax.fori_loop(0, CH, edge_bodyA, 0, unroll=4)

    issueA(0, 0)

    def pairA(p, carry):
        c0 = p * 2
        issueA(c0 + 1, 1)
        computeA(c0, 0)

        @pl.when(c0 + 2 < NCHUNK)
        def _():
            issueA(c0 + 2, 0)
        computeA(c0 + 1, 1)
        return carry

    lax.fori_loop(0, NCHUNK // 2, pairA, 0)

    pB = [ic0_v, ic1_v]
    vB = [f0_v, f1_v]
    msg = [f4_v, f5_v]
    zbuf = f2_v
    for r in range(CH):
        zbuf[r] = zero16

    def issueB(c, b, bb):
        for g in range(CH // 16):
            sv = sall_v[pl.ds(c * CH + g * 16, 16)]
            pB[b][pl.ds(g * 16, 16)] = sv + bb * N_NODE
        pltpu.async_copy(vrelp_hbm.at[pB[b]], vB[b], s0[b])

    def computeB(c, b, bb, first):
        pltpu.make_async_copy(vrelp_hbm.at[pB[b]], vB[b], s0[b]).wait()
        didx = dall_v.at[pl.ds(c * CH, CH)]
        if not first:
            pltpu.make_async_copy(msg[b], out_sp.at[didx], s1[b]).wait()
        h_of_bb = bb >> 1

        def edge_bodyB(r, carry):
            aaddr = jnp.full((16,), (c * CH + r) * 4 + h_of_bb, jnp.int32)
            av = plsc.load_gather(a_sp, [aaddr])
            msg[b][r] = vB[b][r] * av
            return carry

        lax.fori_loop(0, CH, edge_bodyB, 0, unroll=8)
        pltpu.async_copy(msg[b], out_sp.at[didx], s1[b], add=True)

    def head_block(bb, carry):
        for j in range(RPT // CH):
            pltpu.sync_copy(zbuf, out_sp.at[pl.ds(row0 + j * CH, CH), :])
        rem = RPT - (RPT // CH) * CH
        if rem:
            pltpu.sync_copy(zbuf.at[pl.ds(0, rem), :],
                            out_sp.at[pl.ds(row0 + (RPT // CH) * CH, rem), :])
        plsc.subcore_barrier()

        issueB(0, 0, bb)
        issueB(1, 1, bb)
        computeB(0, 0, bb, True)
        issueB(2, 0, bb)
        computeB(1, 1, bb, True)

        def pairB(p, carry2):
            c0 = p * 2
            issueB(c0 + 3, 1, bb)
            computeB(c0 + 2, 0, bb, False)

            @pl.when(c0 + 4 < NCHUNK)
            def _():
                issueB(c0 + 4, 0, bb)
            computeB(c0 + 3, 1, bb, False)
            return carry2

        lax.fori_loop(0, NCHUNK // 2 - 1, pairB, 0)
        dd0 = dall_v.at[pl.ds((NCHUNK - 2) * CH, CH)]
        dd1 = dall_v.at[pl.ds((NCHUNK - 1) * CH, CH)]
        pltpu.make_async_copy(msg[0], out_sp.at[dd0], s1[0]).wait()
        pltpu.make_async_copy(msg[1], out_sp.at[dd1], s1[1]).wait()
        plsc.subcore_barrier()

        @pl.when(cid == 0)
        def _():
            pltpu.sync_copy(out_sp.at[pl.ds(row0, RPT), :],
                            out0_hbm.at[pl.ds(bb * NSP + row0, RPT), :])

        @pl.when(cid == 1)
        def _():
            pltpu.sync_copy(out_sp.at[pl.ds(row0, RPT), :],
                            out1_hbm.at[pl.ds(bb * NSP + row0, RPT), :])
        plsc.subcore_barrier()
        return carry

    lax.fori_loop(0, 2 * H, head_block, 0)


def _sc_pass2(vrel_pack, s_pad, d_pad, ex, den0, den1):
    mesh = plsc.VectorSubcoreMesh(core_axis_name="c", subcore_axis_name="s",
                                  num_cores=NC, num_subcores=NS)
    fn = pl.kernel(
        _pass2_body,
        out_type=[
            jax.ShapeDtypeStruct((2 * H * NSP, 16), jnp.float32),
            jax.ShapeDtypeStruct((2 * H * NSP, 16), jnp.float32),
        ],
        mesh=mesh,
        compiler_params=pltpu.CompilerParams(
            use_tc_tiling_on_sc=False, needs_layout_passes=False),
        scratch_types=[
            pltpu.VMEM((EPT,), jnp.int32),
            pltpu.VMEM((EPT,), jnp.int32),
            pltpu.VMEM((CH,), jnp.int32),
            pltpu.VMEM((CH,), jnp.int32),
            pltpu.VMEM((CH, 16), jnp.float32),
            pltpu.VMEM((CH, 16), jnp.float32),
            pltpu.VMEM((CH, 16), jnp.float32),
            pltpu.VMEM((CH, 16), jnp.float32),
            pltpu.VMEM((CH, 16), jnp.float32),
            pltpu.VMEM((CH, 16), jnp.float32),
            pltpu.VMEM((EPT * 4,), jnp.float32),
            pltpu.VMEM_SHARED((NSP, 16), jnp.float32),
            pltpu.SemaphoreType.DMA,
            pltpu.SemaphoreType.DMA,
            pltpu.SemaphoreType.DMA,
            pltpu.SemaphoreType.DMA,
        ],
    )
    return fn(vrel_pack, s_pad, d_pad, ex, den0, den1)


def _edge_phase(q_dst, krel_src, vrel_src, s, d, n_dst):
    pad = jnp.arange(E_PAD - E_REAL, dtype=jnp.int32) % N_NODE
    s_pad = jnp.concatenate([s.astype(jnp.int32), pad])
    d_pad = jnp.concatenate([d.astype(jnp.int32), pad])
    ex, den0, den1 = _sc_pass1(q_dst, krel_src, s_pad, d_pad)
    vrel_pack = vrel_src.reshape(N_NODE, 2 * H, 16).transpose(1, 0, 2).reshape(
        2 * H * N_NODE, 16)
    o0, o1 = _sc_pass2(vrel_pack, s_pad, d_pad, ex, den0, den1)
    osum = (o0 + o1).reshape(2 * H, NSP, 16)[:, :n_dst, :]
    return osum.transpose(1, 0, 2).reshape(n_dst, C)


_SRC_EDGE = {'user': 'ui', 'item': 'iu'}
_EDGE_DEFS = (('ui', 'user', 'item'), ('iu', 'item', 'user'))


def _fold_params(params):
    folded = {}
    inv_sqrt = 1.0 / math.sqrt(float(DH))
    for l in range(2):
        for t in ('user', 'item'):
            e = _SRC_EDGE[t]
            arel = params['l%d_arel_%s' % (l, e)]
            mrel = params['l%d_mrel_%s' % (l, e)]
            prel = params['l%d_prel_%s' % (l, e)] * inv_sqrt
            Wk = params['l%d_Wk_%s' % (l, t)].reshape(C, H, DH)
            Wv = params['l%d_Wv_%s' % (l, t)].reshape(C, H, DH)
            Wk_f = jnp.einsum('chd,hde,h->che', Wk, arel, prel).reshape(C, C)
            Wv_f = jnp.einsum('chd,hde->che', Wv, mrel).reshape(C, C)
            Wq = params['l%d_Wq_%s' % (l, t)]
            folded['Wqkv_%d_%s' % (l, t)] = jnp.concatenate([Wq, Wk_f, Wv_f], axis=1)
    return folded


def kernel(x_user, x_item, edge_index_user_item, edge_index_item_user, params):
    folded = _fold_params(params)
    h = {'user': _proj_relu(x_user, params['in_W_user'], params['in_b_user']),
         'item': _proj_relu(x_item, params['in_W_item'], params['in_b_item'])}
    ei = {'ui': (edge_index_user_item[0], edge_index_user_item[1]),
          'iu': (edge_index_item_user[0], edge_index_item_user[1])}
    for l in range(2):
        q, krel, vrel = {}, {}, {}
        for t in h:
            y = _proj(h[t], folded['Wqkv_%d_%s' % (l, t)])
            q[t] = y[:, :C]
            krel[t] = y[:, C:2 * C]
            vrel[t] = y[:, 2 * C:]
        out = {}
        for e, src, dst in _EDGE_DEFS:
            s, d = ei[e]
            out[dst] = _edge_phase(q[dst], krel[src], vrel[src], s, d, N_NODE)
        h_new = {}
        for t in h:
            beta = jax.nn.sigmoid(params['l%d_skip_%s' % (l, t)])
            hcoef = (1.0 - beta) + (1.0 if l > 0 else 0.0)
            h_new[t] = _out_stage(out[t], h[t],
                                  params['l%d_Wa_%s' % (l, t)],
                                  params['l%d_ba_%s' % (l, t)], beta, hcoef)
        h = h_new
    return (h['user'], h['item'])
